# Initial kernel scaffold; baseline (speedup 1.0000x reference)
import functools

import jax
import jax.numpy as jnp
from jax import lax
from jax.experimental import pallas as pl
from jax.experimental.pallas import tpu as pltpu
from jax.experimental.pallas import tpu_sc as plsc

B = 4
N0 = 4096
K_NN = 16

_F32 = jnp.float32
_I32 = jnp.int32


def _dot(a, b):
    return jnp.dot(a, b, preferred_element_type=_F32)


def _dot_nt(a, b):
    return lax.dot_general(a, b, (((1,), (1,)), ((), ())),
                           preferred_element_type=_F32)


def _ln_in(x, g, b, eps=1e-5):
    mu = jnp.mean(x, -1, keepdims=True)
    var = jnp.mean((x - mu) ** 2, -1, keepdims=True)
    return (x - mu) / jnp.sqrt(var + eps) * g + b


def _linear_ln_body(x_ref, w_ref, b_ref, g_ref, bb_ref, o_ref, *, relu_after):
    h = _dot(x_ref[0], w_ref[...]) + b_ref[...]
    h = _ln_in(h, g_ref[...], bb_ref[...])
    if relu_after:
        h = jnp.maximum(h, 0.0)
    o_ref[0] = h


def _linear_ln(x, w, bias, g, bb, relu_after):
    b_, n_, din = x.shape
    dout = w.shape[1]
    bl = min(n_, 2048)
    body = functools.partial(_linear_ln_body, relu_after=relu_after)
    return pl.pallas_call(
        body,
        grid=(b_, n_ // bl),
        in_specs=[
            pl.BlockSpec((1, bl, din), lambda b, i: (b, i, 0)),
            pl.BlockSpec((din, dout), lambda b, i: (0, 0)),
            pl.BlockSpec((1, dout), lambda b, i: (0, 0)),
            pl.BlockSpec((1, dout), lambda b, i: (0, 0)),
            pl.BlockSpec((1, dout), lambda b, i: (0, 0)),
        ],
        out_specs=pl.BlockSpec((1, bl, dout), lambda b, i: (b, i, 0)),
        out_shape=jax.ShapeDtypeStruct((b_, n_, dout), _F32),
    )(x, w, bias, g, bb)


def _knn_body(q_ref, t_ref, o_ref, *, k, nt):
    b = pl.program_id(0)
    q = q_ref[0]
    t = t_ref[0]
    qt = _dot_nt(q, t)
    qsq = jnp.sum(q * q, axis=1, keepdims=True)
    tsq_row = _dot_nt(jnp.ones((1, q.shape[1]), _F32), t * t)
    d2 = (qsq + tsq_row) - 2.0 * qt
    cio = lax.broadcasted_iota(_I32, d2.shape, 1)
    big = jnp.int32(2 ** 30)
    cols = []
    for _ in range(k):
        mn = jnp.min(d2, axis=1, keepdims=True)
        cand = jnp.where(d2 <= mn, cio, big)
        idx = jnp.min(cand, axis=1, keepdims=True)
        cols.append(idx)
        d2 = jnp.where(cio == idx, jnp.float32(jnp.inf), d2)
    o_ref[0] = jnp.concatenate(cols, axis=1) + b * nt


def _knn(qxyz, txyz, k):
    b_, nq, _ = qxyz.shape
    nt = txyz.shape[1]
    bq = min(nq, 512)
    body = functools.partial(_knn_body, k=k, nt=nt)
    return pl.pallas_call(
        body,
        grid=(b_, nq // bq),
        in_specs=[
            pl.BlockSpec((1, bq, 16), lambda b, i: (b, i, 0)),
            pl.BlockSpec((1, nt, 16), lambda b, i: (b, 0, 0)),
        ],
        out_specs=pl.BlockSpec((1, bq, k), lambda b, i: (b, i, 0)),
        out_shape=jax.ShapeDtypeStruct((b_, nq, k), _I32),
    )(qxyz, txyz)


def _fps_body(x_ref, y_ref, z_ref, o_ref, *, npoint, n, r):
    b = pl.program_id(0)
    xs = x_ref[0]
    ys = y_ref[0]
    zs = z_ref[0]
    lin = (lax.broadcasted_iota(_I32, (r, 128), 0) * 128
           + lax.broadcasted_iota(_I32, (r, 128), 1))
    big = jnp.int32(2 ** 30)

    def red2(x, fn):
        return fn(fn(x, axis=1, keepdims=True), axis=0, keepdims=True)

    def body(t, carry):
        dist, far = carry
        o_ref[0, pl.ds(t, 1), :] = far + b * n
        m1 = lin == far
        cx = red2(jnp.where(m1, xs, 0.0), jnp.sum)
        cy = red2(jnp.where(m1, ys, 0.0), jnp.sum)
        cz = red2(jnp.where(m1, zs, 0.0), jnp.sum)
        dx = xs - cx
        dy = ys - cy
        dz = zs - cz
        d2 = (dx * dx + dy * dy) + dz * dz
        dist = jnp.minimum(dist, d2)
        mx = red2(dist, jnp.max)
        far = red2(jnp.where(dist >= mx, lin, big), jnp.min)
        return dist, far

    dist0 = jnp.full((r, 128), 1e10, _F32)
    far0 = jnp.zeros((1, 1), _I32)
    lax.fori_loop(0, npoint, body, (dist0, far0))


def _fps(xyz, npoint):
    b_, n_, _ = xyz.shape
    r = n_ // 128
    xs = xyz[..., 0].reshape(b_, r, 128)
    ys = xyz[..., 1].reshape(b_, r, 128)
    zs = xyz[..., 2].reshape(b_, r, 128)
    body = functools.partial(_fps_body, npoint=npoint, n=n_, r=r)
    out = pl.pallas_call(
        body,
        grid=(b_,),
        in_specs=[pl.BlockSpec((1, r, 128), lambda b: (b, 0, 0))] * 3,
        out_specs=pl.BlockSpec((1, npoint, 1), lambda b: (b, 0, 0)),
        out_shape=jax.ShapeDtypeStruct((b_, npoint, 1), _I32),
    )(xs, ys, zs)
    return out.reshape(b_ * npoint)


def _gather(table, idx):
    m = idx.shape[0]
    d = table.shape[1]
    info = plsc.get_sparse_core_info()
    nw = info.num_cores * info.num_subcores
    rows_pw = m // nw
    chunk = min(128, rows_pw)
    n_chunks = rows_pw // chunk
    mesh = plsc.VectorSubcoreMesh(core_axis_name="c", subcore_axis_name="s")

    @functools.partial(
        pl.kernel, mesh=mesh,
        out_type=jax.ShapeDtypeStruct((m, d), _F32),
        scratch_types=[
            pltpu.VMEM((rows_pw,), _I32),
            pltpu.VMEM((2, chunk, d), _F32),
            pltpu.SemaphoreType.DMA,
            pltpu.SemaphoreType.DMA,
        ],
    )
    def k(table_hbm, idx_hbm, out_hbm, idx_v, rows_v, sem0, sem1):
        wid = lax.axis_index("s") * info.num_cores + lax.axis_index("c")
        base = wid * rows_pw
        pltpu.sync_copy(idx_hbm.at[pl.ds(base, rows_pw)], idx_v)
        sems = (sem0, sem1)

        def start(j, slot):
            pltpu.async_copy(
                table_hbm.at[idx_v.at[pl.ds(j * chunk, chunk)]],
                rows_v.at[slot], sems[slot])

        def wait(j, slot):
            pltpu.make_async_copy(
                table_hbm.at[idx_v.at[pl.ds(j * chunk, chunk)]],
                rows_v.at[slot], sems[slot]).wait()

        def store(j, slot):
            pltpu.sync_copy(rows_v.at[slot],
                            out_hbm.at[pl.ds(base + j * chunk, chunk)])

        if n_chunks == 1:
            start(0, 0)
            wait(0, 0)
            store(0, 0)
        else:
            start(0, 0)
            start(1, 1)

            def pair(g, carry):
                for slot in range(2):
                    j = g * 2 + slot
                    wait(j, slot)
                    store(j, slot)

                    @pl.when(j + 2 < n_chunks)
                    def _():
                        start(j + 2, slot)
                return carry

            lax.fori_loop(0, n_chunks // 2, pair, 0)

    return k(table, idx)


def _pt_body(f_ref, xp_ref, gx_ref, gf_ref, wq_ref, wk_ref, wv_ref,
             p1_ref, p1b_ref, p2_ref, p2b_ref, a1_ref, a1b_ref,
             a2_ref, a2b_ref, g_ref, b_ref, o_ref, *, bn, k, dim):
    f = f_ref[0]
    xp = xp_ref[0]
    gx = gx_ref[...]
    gf = gf_ref[...]
    q3 = _dot(f, wq_ref[...])[:, None, :]
    kk = _dot(gf, wk_ref[...]).reshape(bn, k, dim)
    v = _dot(gf, wv_ref[...]).reshape(bn, k, dim)
    pe_own = _dot(xp, p1_ref[...])[:, None, :]
    pe_nb = _dot(gx, p1_ref[...]).reshape(bn, k, dim)
    peh = jnp.maximum(pe_own - pe_nb + p1b_ref[...][None], 0.0)
    pe = (_dot(peh.reshape(bn * k, dim), p2_ref[...])
          + p2b_ref[...]).reshape(bn, k, dim)
    s = q3 - kk + pe
    h = jnp.maximum(_dot(s.reshape(bn * k, dim), a1_ref[...])
                    + a1b_ref[...], 0.0)
    a = (_dot(h, a2_ref[...]) + a2b_ref[...]).reshape(bn, k, dim)
    mx = jnp.max(a, axis=1, keepdims=True)
    e = jnp.exp(a - mx)
    attn = e / jnp.sum(e, axis=1, keepdims=True)
    out = jnp.sum((v + pe) * attn, axis=1)
    o_ref[0] = _ln_in(out + f, g_ref[...], b_ref[...])


def _pt(feat, xp, gx, gf, p, k):
    b_, n_, dim = feat.shape
    bn = min(n_, 256)
    nb = n_ // bn
    body = functools.partial(_pt_body, bn=bn, k=k, dim=dim)
    row2 = lambda b, i: (b * nb + i, 0)
    wspec = pl.BlockSpec((dim, dim), lambda b, i: (0, 0))
    bspec = pl.BlockSpec((1, dim), lambda b, i: (0, 0))
    return pl.pallas_call(
        body,
        grid=(b_, nb),
        in_specs=[
            pl.BlockSpec((1, bn, dim), lambda b, i: (b, i, 0)),
            pl.BlockSpec((1, bn, 16), lambda b, i: (b, i, 0)),
            pl.BlockSpec((bn * k, 16), row2),
            pl.BlockSpec((bn * k, dim), row2),
            wspec, wspec, wspec,
            pl.BlockSpec((16, dim), lambda b, i: (0, 0)), bspec,
            wspec, bspec,
            wspec, bspec,
            wspec, bspec,
            bspec, bspec,
        ],
        out_specs=pl.BlockSpec((1, bn, dim), lambda b, i: (b, i, 0)),
        out_shape=jax.ShapeDtypeStruct((b_, n_, dim), _F32),
    )(feat, xp, gx, gf, p['wq'], p['wk'], p['wv'],
      jnp.pad(p['pe1_w'], ((0, 13), (0, 0))), p['pe1_b'][None],
      p['pe2_w'], p['pe2_b'][None],
      p['am1_w'], p['am1_b'][None],
      p['am2_w'], p['am2_b'][None],
      p['ln_g'][None], p['ln_b'][None])


def _td_body(nx_ref, gx_ref, gf_ref, w1x_ref, w1f_ref, b1_ref,
             w2_ref, b2_ref, g_ref, b_ref, o_ref, *, bn, k, dout):
    nx = nx_ref[0]
    gx = gx_ref[...]
    gf = gf_ref[...]
    hx = _dot(gx, w1x_ref[...]).reshape(bn, k, dout)
    ox = _dot(nx, w1x_ref[...])[:, None, :]
    hf = _dot(gf, w1f_ref[...]).reshape(bn, k, dout)
    h1 = jnp.maximum(hx - ox + hf + b1_ref[...][None], 0.0)
    h2 = (_dot(h1.reshape(bn * k, dout), w2_ref[...])
          + b2_ref[...]).reshape(bn, k, dout)
    nf = jnp.max(h2, axis=1)
    o_ref[0] = _ln_in(nf, g_ref[...], b_ref[...])


def _td(nxp, gx, gf, p, k):
    b_, np_, _ = nxp.shape
    din = gf.shape[1]
    dout = p['w2'].shape[0]
    bn = min(np_, 256)
    nb = np_ // bn
    body = functools.partial(_td_body, bn=bn, k=k, dout=dout)
    row2 = lambda b, i: (b * nb + i, 0)
    w1x = jnp.pad(p['w1'][:3], ((0, 13), (0, 0)))
    w1f = p['w1'][3:]
    return pl.pallas_call(
        body,
        grid=(b_, nb),
        in_specs=[
            pl.BlockSpec((1, bn, 16), lambda b, i: (b, i, 0)),
            pl.BlockSpec((bn * k, 16), row2),
            pl.BlockSpec((bn * k, din), row2),
            pl.BlockSpec((16, dout), lambda b, i: (0, 0)),
            pl.BlockSpec((din, dout), lambda b, i: (0, 0)),
            pl.BlockSpec((1, dout), lambda b, i: (0, 0)),
            pl.BlockSpec((dout, dout), lambda b, i: (0, 0)),
            pl.BlockSpec((1, dout), lambda b, i: (0, 0)),
            pl.BlockSpec((1, dout), lambda b, i: (0, 0)),
            pl.BlockSpec((1, dout), lambda b, i: (0, 0)),
        ],
        out_specs=pl.BlockSpec((1, bn, dout), lambda b, i: (b, i, 0)),
        out_shape=jax.ShapeDtypeStruct((b_, np_, dout), _F32),
    )(nxp, gx, gf, w1x, w1f, p['b1'][None], p['w2'], p['b2'][None],
      p['ln_g'][None], p['ln_b'][None])


def _tu_body(hx_ref, lx_ref, fs_ref, fl_ref, w1s_ref, w1i_ref, b1_ref,
             w2_ref, b2_ref, g_ref, b_ref, o_ref):
    q = hx_ref[0]
    t = lx_ref[0]
    qt = _dot_nt(q, t)
    qsq = jnp.sum(q * q, axis=1, keepdims=True)
    tsq_row = _dot_nt(jnp.ones((1, q.shape[1]), _F32), t * t)
    d2 = (qsq + tsq_row) - 2.0 * qt
    cio = lax.broadcasted_iota(_I32, d2.shape, 1)
    big = jnp.int32(2 ** 30)
    w = jnp.zeros(d2.shape, _F32)
    invsum = jnp.zeros((d2.shape[0], 1), _F32)
    for _ in range(3):
        mn = jnp.min(d2, axis=1, keepdims=True)
        cand = jnp.where(d2 <= mn, cio, big)
        idx = jnp.min(cand, axis=1, keepdims=True)
        dj = jnp.sqrt(jnp.maximum(mn, 0.0))
        invj = 1.0 / (dj + 1e-8)
        sel = cio == idx
        w = w + jnp.where(sel, invj, 0.0)
        invsum = invsum + invj
        d2 = jnp.where(sel, jnp.float32(jnp.inf), d2)
    w = w / invsum
    interp = _dot(w, fl_ref[0])
    h = jnp.maximum(_dot(fs_ref[0], w1s_ref[...])
                    + _dot(interp, w1i_ref[...]) + b1_ref[...], 0.0)
    h2 = _dot(h, w2_ref[...]) + b2_ref[...]
    o_ref[0] = _ln_in(h2, g_ref[...], b_ref[...])


def _tu(hxp, lxp, fskip, flo, p):
    b_, nhi, _ = hxp.shape
    nlo = lxp.shape[1]
    dskip = fskip.shape[2]
    dlo = flo.shape[2]
    dout = p['w2'].shape[0]
    bn = min(nhi, 256)
    nb = nhi // bn
    return pl.pallas_call(
        _tu_body,
        grid=(b_, nb),
        in_specs=[
            pl.BlockSpec((1, bn, 16), lambda b, i: (b, i, 0)),
            pl.BlockSpec((1, nlo, 16), lambda b, i: (b, 0, 0)),
            pl.BlockSpec((1, bn, dskip), lambda b, i: (b, i, 0)),
            pl.BlockSpec((1, nlo, dlo), lambda b, i: (b, 0, 0)),
            pl.BlockSpec((dskip, dout), lambda b, i: (0, 0)),
            pl.BlockSpec((dlo, dout), lambda b, i: (0, 0)),
            pl.BlockSpec((1, dout), lambda b, i: (0, 0)),
            pl.BlockSpec((dout, dout), lambda b, i: (0, 0)),
            pl.BlockSpec((1, dout), lambda b, i: (0, 0)),
            pl.BlockSpec((1, dout), lambda b, i: (0, 0)),
            pl.BlockSpec((1, dout), lambda b, i: (0, 0)),
        ],
        out_specs=pl.BlockSpec((1, bn, dout), lambda b, i: (b, i, 0)),
        out_shape=jax.ShapeDtypeStruct((b_, nhi, dout), _F32),
    )(hxp, lxp, fskip, flo, p['w1'][:dskip], p['w1'][dskip:], p['b1'][None],
      p['w2'], p['b2'][None], p['ln_g'][None], p['ln_b'][None])


def kernel(x, params):
    p = params
    b_, n_, _ = x.shape
    xyz = x[..., :3]
    xp0 = jnp.pad(xyz, ((0, 0), (0, 0), (0, 13)))
    x8 = jnp.pad(x, ((0, 0), (0, 0), (0, 2)))
    stem_w = jnp.pad(p['stem_w'], ((0, 2), (0, 0)))
    f0 = _linear_ln(x8, stem_w, p['stem_b'][None],
                    p['stem_ln_g'][None], p['stem_ln_b'][None], True)

    xt0 = xp0.reshape(b_ * n_, 16)
    idx0 = _knn(xp0, xp0, K_NN).reshape(-1)
    gx0 = _gather(xt0, idx0)
    gf0 = _gather(f0.reshape(-1, 64), idx0)
    f0p = _pt(f0, xp0, gx0, gf0, p['pt0'], K_NN)

    fi1 = _fps(xyz, 1024)
    xp1 = _gather(xt0, fi1).reshape(b_, 1024, 16)
    idxd1 = _knn(xp1, xp0, K_NN).reshape(-1)
    gxd1 = _gather(xt0, idxd1)
    gfd1 = _gather(f0p.reshape(-1, 64), idxd1)
    f1 = _td(xp1, gxd1, gfd1, p['td1'], K_NN)

    xt1 = xp1.reshape(-1, 16)
    idx1 = _knn(xp1, xp1, K_NN).reshape(-1)
    gx1 = _gather(xt1, idx1)
    gf1 = _gather(f1.reshape(-1, 128), idx1)
    f1p = _pt(f1, xp1, gx1, gf1, p['pt1'], K_NN)

    fi2 = _fps(xp1[..., :3], 256)
    xp2 = _gather(xt1, fi2).reshape(b_, 256, 16)
    idxd2 = _knn(xp2, xp1, K_NN).reshape(-1)
    gxd2 = _gather(xt1, idxd2)
    gfd2 = _gather(f1p.reshape(-1, 128), idxd2)
    f2 = _td(xp2, gxd2, gfd2, p['td2'], K_NN)

    xt2 = xp2.reshape(-1, 16)
    idx2 = _knn(xp2, xp2, K_NN).reshape(-1)
    gx2 = _gather(xt2, idx2)
    gf2 = _gather(f2.reshape(-1, 256), idx2)
    f2p = _pt(f2, xp2, gx2, gf2, p['pt2'], K_NN)

    f1u = _tu(xp1, xp2, f1p, f2p, p['tu1'])
    gf1u = _gather(f1u.reshape(-1, 128), idx1)
    f1d = _pt(f1u, xp1, gx1, gf1u, p['ptd1'], K_NN)

    f0u = _tu(xp0, xp1, f0p, f1d, p['tu2'])
    gf0u = _gather(f0u.reshape(-1, 64), idx0)
    f0d = _pt(f0u, xp0, gx0, gf0u, p['ptd2'], K_NN)

    return _linear_ln(f0d, p['head_w'], p['head_b'][None],
                      p['head_ln_g'][None], p['head_ln_b'][None], False)


# speedup vs baseline: 8.5478x; 8.5478x over previous
import functools

import jax
import jax.numpy as jnp
from jax import lax
from jax.experimental import pallas as pl
from jax.experimental.pallas import tpu as pltpu
from jax.experimental.pallas import tpu_sc as plsc

B = 4
N0 = 4096
K_NN = 16

_F32 = jnp.float32
_I32 = jnp.int32


def _dot(a, b):
    return jnp.dot(a, b, preferred_element_type=_F32)


def _dot_nt(a, b):
    return lax.dot_general(a, b, (((1,), (1,)), ((), ())),
                           preferred_element_type=_F32)


def _dot_hi(a, b):
    return jnp.dot(a, b, preferred_element_type=_F32,
                   precision=lax.Precision.HIGHEST)


def _pair_dist(q, t):
    qt = _dot_nt(q, t)
    qsq = jnp.sum(q * q, axis=1, keepdims=True)
    tsq_row = lax.dot_general(jnp.ones((1, q.shape[1]), _F32), t * t,
                              (((1,), (1,)), ((), ())),
                              preferred_element_type=_F32,
                              precision=lax.Precision.HIGHEST)
    d2 = (qsq + tsq_row) - 2.0 * qt
    return jnp.sqrt(jnp.maximum(d2, 0.0))


def _ln_in(x, g, b, eps=1e-5):
    mu = jnp.mean(x, -1, keepdims=True)
    var = jnp.mean((x - mu) ** 2, -1, keepdims=True)
    return (x - mu) / jnp.sqrt(var + eps) * g + b


def _linear_ln_body(x_ref, w_ref, b_ref, g_ref, bb_ref, o_ref, *, relu_after):
    h = _dot(x_ref[0], w_ref[...]) + b_ref[...]
    h = _ln_in(h, g_ref[...], bb_ref[...])
    if relu_after:
        h = jnp.maximum(h, 0.0)
    o_ref[0] = h


def _linear_ln(x, w, bias, g, bb, relu_after):
    b_, n_, din = x.shape
    dout = w.shape[1]
    bl = min(n_, 2048)
    body = functools.partial(_linear_ln_body, relu_after=relu_after)
    return pl.pallas_call(
        body,
        grid=(b_, n_ // bl),
        in_specs=[
            pl.BlockSpec((1, bl, din), lambda b, i: (b, i, 0)),
            pl.BlockSpec((din, dout), lambda b, i: (0, 0)),
            pl.BlockSpec((1, dout), lambda b, i: (0, 0)),
            pl.BlockSpec((1, dout), lambda b, i: (0, 0)),
            pl.BlockSpec((1, dout), lambda b, i: (0, 0)),
        ],
        out_specs=pl.BlockSpec((1, bl, dout), lambda b, i: (b, i, 0)),
        out_shape=jax.ShapeDtypeStruct((b_, n_, dout), _F32),
    )(x, w, bias, g, bb)


def _knn_body(q_ref, t_ref, o_ref, *, k, nt):
    b = pl.program_id(0)
    q = q_ref[0]
    t = t_ref[0]
    dd = _pair_dist(q, t)
    cio = lax.broadcasted_iota(_I32, dd.shape, 1)
    big = jnp.int32(2 ** 30)
    cols = []
    for _ in range(k):
        mn = jnp.min(dd, axis=1, keepdims=True)
        cand = jnp.where(dd <= mn, cio, big)
        idx = jnp.min(cand, axis=1, keepdims=True)
        cols.append(idx)
        dd = jnp.where(cio == idx, jnp.float32(jnp.inf), dd)
    o_ref[0] = jnp.concatenate(cols, axis=1) + b * nt


def _knn(qxyz, txyz, k):
    b_, nq, _ = qxyz.shape
    nt = txyz.shape[1]
    bq = min(nq, 512)
    body = functools.partial(_knn_body, k=k, nt=nt)
    return pl.pallas_call(
        body,
        grid=(b_, nq // bq),
        in_specs=[
            pl.BlockSpec((1, bq, 16), lambda b, i: (b, i, 0)),
            pl.BlockSpec((1, nt, 16), lambda b, i: (b, 0, 0)),
        ],
        out_specs=pl.BlockSpec((1, bq, k), lambda b, i: (b, i, 0)),
        out_shape=jax.ShapeDtypeStruct((b_, nq, k), _I32),
    )(qxyz, txyz)


def _fps_body(x_ref, y_ref, z_ref, o_ref, *, npoint, n, r):
    b = pl.program_id(0)
    xs = x_ref[0]
    ys = y_ref[0]
    zs = z_ref[0]
    lin = (lax.broadcasted_iota(_I32, (r, 128), 0) * 128
           + lax.broadcasted_iota(_I32, (r, 128), 1))
    big = jnp.int32(2 ** 30)

    def red2(x, fn):
        return fn(fn(x, axis=1, keepdims=True), axis=0, keepdims=True)

    def body(t, carry):
        dist, far = carry
        o_ref[0, pl.ds(t, 1), :] = far + b * n
        m1 = lin == far
        cx = red2(jnp.where(m1, xs, 0.0), jnp.sum)
        cy = red2(jnp.where(m1, ys, 0.0), jnp.sum)
        cz = red2(jnp.where(m1, zs, 0.0), jnp.sum)
        dx = xs - cx
        dy = ys - cy
        dz = zs - cz
        d2 = (dx * dx + dy * dy) + dz * dz
        dist = jnp.minimum(dist, d2)
        mx = red2(dist, jnp.max)
        far = red2(jnp.where(dist >= mx, lin, big), jnp.min)
        return dist, far

    dist0 = jnp.full((r, 128), 1e10, _F32)
    far0 = jnp.zeros((1, 1), _I32)
    lax.fori_loop(0, npoint, body, (dist0, far0))


def _fps(xyz, npoint):
    b_, n_, _ = xyz.shape
    r = n_ // 128
    xs = xyz[..., 0].reshape(b_, r, 128)
    ys = xyz[..., 1].reshape(b_, r, 128)
    zs = xyz[..., 2].reshape(b_, r, 128)
    body = functools.partial(_fps_body, npoint=npoint, n=n_, r=r)
    out = pl.pallas_call(
        body,
        grid=(b_,),
        in_specs=[pl.BlockSpec((1, r, 128), lambda b: (b, 0, 0))] * 3,
        out_specs=pl.BlockSpec((1, npoint, 1), lambda b: (b, 0, 0)),
        out_shape=jax.ShapeDtypeStruct((b_, npoint, 1), _I32),
    )(xs, ys, zs)
    return out.reshape(b_ * npoint)


def _gather(table, idx):
    m = idx.shape[0]
    d = table.shape[1]
    info = plsc.get_sparse_core_info()
    nw = info.num_cores * info.num_subcores
    rows_pw = m // nw
    chunk = min(128, rows_pw)
    n_chunks = rows_pw // chunk
    mesh = plsc.VectorSubcoreMesh(core_axis_name="c", subcore_axis_name="s")

    @functools.partial(
        pl.kernel, mesh=mesh,
        out_type=jax.ShapeDtypeStruct((m, d), _F32),
        scratch_types=[
            pltpu.VMEM((rows_pw,), _I32),
            pltpu.VMEM((2, chunk, d), _F32),
            pltpu.SemaphoreType.DMA,
            pltpu.SemaphoreType.DMA,
        ],
    )
    def k(table_hbm, idx_hbm, out_hbm, idx_v, rows_v, sem0, sem1):
        wid = lax.axis_index("s") * info.num_cores + lax.axis_index("c")
        base = wid * rows_pw
        pltpu.sync_copy(idx_hbm.at[pl.ds(base, rows_pw)], idx_v)
        sems = (sem0, sem1)

        def start(j, slot):
            pltpu.async_copy(
                table_hbm.at[idx_v.at[pl.ds(j * chunk, chunk)]],
                rows_v.at[slot], sems[slot])

        def wait(j, slot):
            pltpu.make_async_copy(
                table_hbm.at[idx_v.at[pl.ds(j * chunk, chunk)]],
                rows_v.at[slot], sems[slot]).wait()

        def store(j, slot):
            pltpu.sync_copy(rows_v.at[slot],
                            out_hbm.at[pl.ds(base + j * chunk, chunk)])

        if n_chunks == 1:
            start(0, 0)
            wait(0, 0)
            store(0, 0)
        else:
            start(0, 0)
            start(1, 1)

            def pair(g, carry):
                for slot in range(2):
                    j = g * 2 + slot
                    wait(j, slot)
                    store(j, slot)

                    @pl.when(j + 2 < n_chunks)
                    def _():
                        start(j + 2, slot)
                return carry

            lax.fori_loop(0, n_chunks // 2, pair, 0)

    return k(table, idx)


def _pt_body(f_ref, xp_ref, gg_ref, wq_ref, wkc_ref, wvc_ref,
             p1o_ref, p1c_ref, p1b_ref, p2_ref, p2b_ref, a1_ref, a1b_ref,
             a2_ref, a2b_ref, g_ref, b_ref, o_ref, *, bn, k, dim):
    f = f_ref[0]
    xp = xp_ref[0]
    gg = gg_ref[...]
    q3 = _dot(f, wq_ref[...])[:, None, :]
    kk = _dot(gg, wkc_ref[...]).reshape(bn, k, dim)
    v = _dot(gg, wvc_ref[...]).reshape(bn, k, dim)
    pe_own = _dot(xp, p1o_ref[...])[:, None, :]
    pe_nb = _dot(gg, p1c_ref[...]).reshape(bn, k, dim)
    peh = jnp.maximum(pe_own - pe_nb + p1b_ref[...][None], 0.0)
    pe = (_dot(peh.reshape(bn * k, dim), p2_ref[...])
          + p2b_ref[...]).reshape(bn, k, dim)
    s = q3 - kk + pe
    h = jnp.maximum(_dot(s.reshape(bn * k, dim), a1_ref[...])
                    + a1b_ref[...], 0.0)
    a = (_dot(h, a2_ref[...]) + a2b_ref[...]).reshape(bn, k, dim)
    mx = jnp.max(a, axis=1, keepdims=True)
    e = jnp.exp(a - mx)
    attn = e / jnp.sum(e, axis=1, keepdims=True)
    out = jnp.sum((v + pe) * attn, axis=1)
    o_ref[0] = _ln_in(out + f, g_ref[...], b_ref[...])


def _pt(feat, xp, gg, p, k):
    b_, n_, dim = feat.shape
    dg = gg.shape[1]
    bn = min(n_, 256)
    nb = n_ // bn
    body = functools.partial(_pt_body, bn=bn, k=k, dim=dim)
    row2 = lambda b, i: (b * nb + i, 0)
    wspec = pl.BlockSpec((dim, dim), lambda b, i: (0, 0))
    cspec = pl.BlockSpec((dg, dim), lambda b, i: (0, 0))
    bspec = pl.BlockSpec((1, dim), lambda b, i: (0, 0))
    z = jnp.zeros((dg, dim), _F32)
    wkc = z.at[16:16 + dim].set(p['wk'])
    wvc = z.at[16:16 + dim].set(p['wv'])
    p1c = z.at[:3].set(p['pe1_w'])
    p1o = jnp.pad(p['pe1_w'], ((0, 13), (0, 0)))
    return pl.pallas_call(
        body,
        grid=(b_, nb),
        in_specs=[
            pl.BlockSpec((1, bn, dim), lambda b, i: (b, i, 0)),
            pl.BlockSpec((1, bn, 16), lambda b, i: (b, i, 0)),
            pl.BlockSpec((bn * k, dg), row2),
            wspec, cspec, cspec,
            pl.BlockSpec((16, dim), lambda b, i: (0, 0)), cspec, bspec,
            wspec, bspec,
            wspec, bspec,
            wspec, bspec,
            bspec, bspec,
        ],
        out_specs=pl.BlockSpec((1, bn, dim), lambda b, i: (b, i, 0)),
        out_shape=jax.ShapeDtypeStruct((b_, n_, dim), _F32),
    )(feat, xp, gg, p['wq'], wkc, wvc,
      p1o, p1c, p['pe1_b'][None],
      p['pe2_w'], p['pe2_b'][None],
      p['am1_w'], p['am1_b'][None],
      p['am2_w'], p['am2_b'][None],
      p['ln_g'][None], p['ln_b'][None])


def _td_body(nx_ref, gg_ref, w1c_ref, w1x_ref, b1_ref,
             w2_ref, b2_ref, g_ref, b_ref, o_ref, *, bn, k, dout):
    nx = nx_ref[0]
    gg = gg_ref[...]
    hg = _dot(gg, w1c_ref[...]).reshape(bn, k, dout)
    ox = _dot(nx, w1x_ref[...])[:, None, :]
    h1 = jnp.maximum(hg - ox + b1_ref[...][None], 0.0)
    h2 = (_dot(h1.reshape(bn * k, dout), w2_ref[...])
          + b2_ref[...]).reshape(bn, k, dout)
    nf = jnp.max(h2, axis=1)
    o_ref[0] = _ln_in(nf, g_ref[...], b_ref[...])


def _td(nxp, gg, p, k, din):
    b_, np_, _ = nxp.shape
    dg = gg.shape[1]
    dout = p['w2'].shape[0]
    bn = min(np_, 256)
    nb = np_ // bn
    body = functools.partial(_td_body, bn=bn, k=k, dout=dout)
    row2 = lambda b, i: (b * nb + i, 0)
    w1x = jnp.pad(p['w1'][:3], ((0, 13), (0, 0)))
    w1c = jnp.zeros((dg, dout), _F32)
    w1c = w1c.at[:3].set(p['w1'][:3]).at[16:16 + din].set(p['w1'][3:])
    return pl.pallas_call(
        body,
        grid=(b_, nb),
        in_specs=[
            pl.BlockSpec((1, bn, 16), lambda b, i: (b, i, 0)),
            pl.BlockSpec((bn * k, dg), row2),
            pl.BlockSpec((dg, dout), lambda b, i: (0, 0)),
            pl.BlockSpec((16, dout), lambda b, i: (0, 0)),
            pl.BlockSpec((1, dout), lambda b, i: (0, 0)),
            pl.BlockSpec((dout, dout), lambda b, i: (0, 0)),
            pl.BlockSpec((1, dout), lambda b, i: (0, 0)),
            pl.BlockSpec((1, dout), lambda b, i: (0, 0)),
            pl.BlockSpec((1, dout), lambda b, i: (0, 0)),
        ],
        out_specs=pl.BlockSpec((1, bn, dout), lambda b, i: (b, i, 0)),
        out_shape=jax.ShapeDtypeStruct((b_, np_, dout), _F32),
    )(nxp, gg, w1c, w1x, p['b1'][None], p['w2'], p['b2'][None],
      p['ln_g'][None], p['ln_b'][None])


def _tu_body(hx_ref, lx_ref, fs_ref, fl_ref, w1s_ref, w1i_ref, b1_ref,
             w2_ref, b2_ref, g_ref, b_ref, o_ref):
    q = hx_ref[0]
    t = lx_ref[0]
    dd = _pair_dist(q, t)
    cio = lax.broadcasted_iota(_I32, dd.shape, 1)
    big = jnp.int32(2 ** 30)
    w = jnp.zeros(dd.shape, _F32)
    invsum = jnp.zeros((dd.shape[0], 1), _F32)
    for _ in range(3):
        mn = jnp.min(dd, axis=1, keepdims=True)
        cand = jnp.where(dd <= mn, cio, big)
        idx = jnp.min(cand, axis=1, keepdims=True)
        invj = 1.0 / (mn + 1e-8)
        sel = cio == idx
        w = w + jnp.where(sel, invj, 0.0)
        invsum = invsum + invj
        dd = jnp.where(sel, jnp.float32(jnp.inf), dd)
    w = w / invsum
    interp = _dot_hi(w, fl_ref[0])
    h = jnp.maximum(_dot(fs_ref[0], w1s_ref[...])
                    + _dot(interp, w1i_ref[...]) + b1_ref[...], 0.0)
    h2 = _dot(h, w2_ref[...]) + b2_ref[...]
    o_ref[0] = _ln_in(h2, g_ref[...], b_ref[...])


def _tu(hxp, lxp, fskip, flo, p):
    b_, nhi, _ = hxp.shape
    nlo = lxp.shape[1]
    dskip = fskip.shape[2]
    dlo = flo.shape[2]
    dout = p['w2'].shape[0]
    bn = min(nhi, 256)
    nb = nhi // bn
    return pl.pallas_call(
        _tu_body,
        grid=(b_, nb),
        in_specs=[
            pl.BlockSpec((1, bn, 16), lambda b, i: (b, i, 0)),
            pl.BlockSpec((1, nlo, 16), lambda b, i: (b, 0, 0)),
            pl.BlockSpec((1, bn, dskip), lambda b, i: (b, i, 0)),
            pl.BlockSpec((1, nlo, dlo), lambda b, i: (b, 0, 0)),
            pl.BlockSpec((dskip, dout), lambda b, i: (0, 0)),
            pl.BlockSpec((dlo, dout), lambda b, i: (0, 0)),
            pl.BlockSpec((1, dout), lambda b, i: (0, 0)),
            pl.BlockSpec((dout, dout), lambda b, i: (0, 0)),
            pl.BlockSpec((1, dout), lambda b, i: (0, 0)),
            pl.BlockSpec((1, dout), lambda b, i: (0, 0)),
            pl.BlockSpec((1, dout), lambda b, i: (0, 0)),
        ],
        out_specs=pl.BlockSpec((1, bn, dout), lambda b, i: (b, i, 0)),
        out_shape=jax.ShapeDtypeStruct((b_, nhi, dout), _F32),
    )(hxp, lxp, fskip, flo, p['w1'][:dskip], p['w1'][dskip:], p['b1'][None],
      p['w2'], p['b2'][None], p['ln_g'][None], p['ln_b'][None])


def _table(xp, feat):
    b_, n_, dim = feat.shape
    dg = ((16 + dim + 127) // 128) * 128
    t = jnp.concatenate(
        [xp, feat, jnp.zeros((b_, n_, dg - 16 - dim), _F32)], axis=-1)
    return t.reshape(b_ * n_, dg)


def kernel(x, params):
    p = params
    b_, n_, _ = x.shape
    xyz = x[..., :3]
    xp0 = jnp.pad(xyz, ((0, 0), (0, 0), (0, 13)))
    x8 = jnp.pad(x, ((0, 0), (0, 0), (0, 2)))
    stem_w = jnp.pad(p['stem_w'], ((0, 2), (0, 0)))
    f0 = _linear_ln(x8, stem_w, p['stem_b'][None],
                    p['stem_ln_g'][None], p['stem_ln_b'][None], True)

    idx0 = _knn(xp0, xp0, K_NN).reshape(-1)
    t0 = _table(xp0, f0)
    g0 = _gather(t0, idx0)
    f0p = _pt(f0, xp0, g0, p['pt0'], K_NN)

    fi1 = _fps(xyz, 1024)
    t0p = _table(xp0, f0p)
    xp1 = _gather(t0p, fi1)[:, :16].reshape(b_, 1024, 16)
    idxd1 = _knn(xp1, xp0, K_NN).reshape(-1)
    gd1 = _gather(t0p, idxd1)
    f1 = _td(xp1, gd1, p['td1'], K_NN, 64)

    idx1 = _knn(xp1, xp1, K_NN).reshape(-1)
    g1 = _gather(_table(xp1, f1), idx1)
    f1p = _pt(f1, xp1, g1, p['pt1'], K_NN)

    fi2 = _fps(xp1[..., :3], 256)
    t1p = _table(xp1, f1p)
    xp2 = _gather(t1p, fi2)[:, :16].reshape(b_, 256, 16)
    idxd2 = _knn(xp2, xp1, K_NN).reshape(-1)
    gd2 = _gather(t1p, idxd2)
    f2 = _td(xp2, gd2, p['td2'], K_NN, 128)

    idx2 = _knn(xp2, xp2, K_NN).reshape(-1)
    g2 = _gather(_table(xp2, f2), idx2)
    f2p = _pt(f2, xp2, g2, p['pt2'], K_NN)

    f1u = _tu(xp1, xp2, f1p, f2p, p['tu1'])
    g1u = _gather(_table(xp1, f1u), idx1)
    f1d = _pt(f1u, xp1, g1u, p['ptd1'], K_NN)

    f0u = _tu(xp0, xp1, f0p, f1d, p['tu2'])
    g0u = _gather(_table(xp0, f0u), idx0)
    f0d = _pt(f0u, xp0, g0u, p['ptd2'], K_NN)

    return _linear_ln(f0d, p['head_w'], p['head_b'][None],
                      p['head_ln_g'][None], p['head_ln_b'][None], False)


# speedup vs baseline: 11.3477x; 1.3276x over previous
import functools

import jax
import jax.numpy as jnp
from jax import lax
from jax.experimental import pallas as pl
from jax.experimental.pallas import tpu as pltpu
from jax.experimental.pallas import tpu_sc as plsc

B = 4
N0 = 4096
K_NN = 16

_F32 = jnp.float32
_I32 = jnp.int32


def _dot(a, b):
    return jnp.dot(a, b, preferred_element_type=_F32)


def _dot_nt(a, b):
    return lax.dot_general(a, b, (((1,), (1,)), ((), ())),
                           preferred_element_type=_F32)


def _dot_hi(a, b):
    return jnp.dot(a, b, preferred_element_type=_F32,
                   precision=lax.Precision.HIGHEST)


def _pair_dist(q, t):
    qt = _dot_nt(q, t)
    qsq = jnp.sum(q * q, axis=1, keepdims=True)
    tsq_row = lax.dot_general(jnp.ones((1, q.shape[1]), _F32), t * t,
                              (((1,), (1,)), ((), ())),
                              preferred_element_type=_F32,
                              precision=lax.Precision.HIGHEST)
    d2 = (qsq + tsq_row) - 2.0 * qt
    return jnp.sqrt(jnp.maximum(d2, 0.0))


def _ln_in(x, g, b, eps=1e-5):
    mu = jnp.mean(x, -1, keepdims=True)
    var = jnp.mean((x - mu) ** 2, -1, keepdims=True)
    return (x - mu) / jnp.sqrt(var + eps) * g + b


def _linear_ln_body(x_ref, w_ref, b_ref, g_ref, bb_ref, o_ref, *, relu_after):
    h = _dot(x_ref[0], w_ref[...]) + b_ref[...]
    h = _ln_in(h, g_ref[...], bb_ref[...])
    if relu_after:
        h = jnp.maximum(h, 0.0)
    o_ref[0] = h


def _linear_ln(x, w, bias, g, bb, relu_after):
    b_, n_, din = x.shape
    dout = w.shape[1]
    bl = min(n_, 2048)
    body = functools.partial(_linear_ln_body, relu_after=relu_after)
    return pl.pallas_call(
        body,
        grid=(b_, n_ // bl),
        in_specs=[
            pl.BlockSpec((1, bl, din), lambda b, i: (b, i, 0)),
            pl.BlockSpec((din, dout), lambda b, i: (0, 0)),
            pl.BlockSpec((1, dout), lambda b, i: (0, 0)),
            pl.BlockSpec((1, dout), lambda b, i: (0, 0)),
            pl.BlockSpec((1, dout), lambda b, i: (0, 0)),
        ],
        out_specs=pl.BlockSpec((1, bl, dout), lambda b, i: (b, i, 0)),
        out_shape=jax.ShapeDtypeStruct((b_, n_, dout), _F32),
    )(x, w, bias, g, bb)


def _knn_body(q_ref, t_ref, o_ref, *, k, nt):
    b = pl.program_id(0)
    q = q_ref[0]
    t = t_ref[0]
    dd = _pair_dist(q, t)
    cio = lax.broadcasted_iota(_I32, dd.shape, 1)
    big = jnp.int32(2 ** 30)
    cols = []
    for _ in range(k):
        mn = jnp.min(dd, axis=1, keepdims=True)
        cand = jnp.where(dd <= mn, cio, big)
        idx = jnp.min(cand, axis=1, keepdims=True)
        cols.append(idx)
        dd = jnp.where(cio == idx, jnp.float32(jnp.inf), dd)
    o_ref[0] = jnp.concatenate(cols, axis=1) + b * nt


def _knn(qxyz, txyz, k):
    b_, nq, _ = qxyz.shape
    nt = txyz.shape[1]
    bq = min(nq, 512)
    body = functools.partial(_knn_body, k=k, nt=nt)
    return pl.pallas_call(
        body,
        grid=(b_, nq // bq),
        in_specs=[
            pl.BlockSpec((1, bq, 16), lambda b, i: (b, i, 0)),
            pl.BlockSpec((1, nt, 16), lambda b, i: (b, 0, 0)),
        ],
        out_specs=pl.BlockSpec((1, bq, k), lambda b, i: (b, i, 0)),
        out_shape=jax.ShapeDtypeStruct((b_, nq, k), _I32),
    )(qxyz, txyz)


def _fps_body(x_ref, y_ref, z_ref, o_ref, *, npoint, n, r, nb):
    xs = x_ref[...]
    ys = y_ref[...]
    zs = z_ref[...]
    lin = (lax.broadcasted_iota(_I32, (nb, r, 128), 1) * 128
           + lax.broadcasted_iota(_I32, (nb, r, 128), 2))
    big = jnp.int32(2 ** 30)

    def red2(x, fn):
        return fn(fn(x, axis=2, keepdims=True), axis=1, keepdims=True)

    def body(t, carry):
        dist, far = carry
        for b in range(nb):
            o_ref[b, pl.ds(t, 1), :] = far[b] + b * n
        m1 = lin == far
        cx = red2(jnp.where(m1, xs, 0.0), jnp.sum)
        cy = red2(jnp.where(m1, ys, 0.0), jnp.sum)
        cz = red2(jnp.where(m1, zs, 0.0), jnp.sum)
        dx = xs - cx
        dy = ys - cy
        dz = zs - cz
        d2 = (dx * dx + dy * dy) + dz * dz
        dist = jnp.minimum(dist, d2)
        mx = red2(dist, jnp.max)
        far = red2(jnp.where(dist >= mx, lin, big), jnp.min)
        return dist, far

    dist0 = jnp.full((nb, r, 128), 1e10, _F32)
    far0 = jnp.zeros((nb, 1, 1), _I32)
    lax.fori_loop(0, npoint, body, (dist0, far0))


def _fps(xyz, npoint):
    b_, n_, _ = xyz.shape
    r = n_ // 128
    xs = xyz[..., 0].reshape(b_, r, 128)
    ys = xyz[..., 1].reshape(b_, r, 128)
    zs = xyz[..., 2].reshape(b_, r, 128)
    body = functools.partial(_fps_body, npoint=npoint, n=n_, r=r, nb=b_)
    out = pl.pallas_call(
        body,
        grid=(1,),
        in_specs=[pl.BlockSpec((b_, r, 128), lambda i: (0, 0, 0))] * 3,
        out_specs=pl.BlockSpec((b_, npoint, 1), lambda i: (0, 0, 0)),
        out_shape=jax.ShapeDtypeStruct((b_, npoint, 1), _I32),
    )(xs, ys, zs)
    return out.reshape(b_ * npoint)


def _gather(table, idx):
    m = idx.shape[0]
    d = table.shape[1]
    info = plsc.get_sparse_core_info()
    nw = info.num_cores * info.num_subcores
    rows_pw = m // nw
    chunk = min(128, rows_pw)
    n_chunks = rows_pw // chunk
    mesh = plsc.VectorSubcoreMesh(core_axis_name="c", subcore_axis_name="s")

    @functools.partial(
        pl.kernel, mesh=mesh,
        out_type=jax.ShapeDtypeStruct((m, d), _F32),
        scratch_types=[
            pltpu.VMEM((rows_pw,), _I32),
            pltpu.VMEM((2, chunk, d), _F32),
            pltpu.SemaphoreType.DMA,
            pltpu.SemaphoreType.DMA,
        ],
    )
    def k(table_hbm, idx_hbm, out_hbm, idx_v, rows_v, sem0, sem1):
        wid = lax.axis_index("s") * info.num_cores + lax.axis_index("c")
        base = wid * rows_pw
        pltpu.sync_copy(idx_hbm.at[pl.ds(base, rows_pw)], idx_v)
        sems = (sem0, sem1)

        def start(j, slot):
            pltpu.async_copy(
                table_hbm.at[idx_v.at[pl.ds(j * chunk, chunk)]],
                rows_v.at[slot], sems[slot])

        def wait(j, slot):
            pltpu.make_async_copy(
                table_hbm.at[idx_v.at[pl.ds(j * chunk, chunk)]],
                rows_v.at[slot], sems[slot]).wait()

        def store(j, slot):
            pltpu.sync_copy(rows_v.at[slot],
                            out_hbm.at[pl.ds(base + j * chunk, chunk)])

        if n_chunks == 1:
            start(0, 0)
            wait(0, 0)
            store(0, 0)
        else:
            start(0, 0)
            start(1, 1)

            def pair(g, carry):
                for slot in range(2):
                    j = g * 2 + slot
                    wait(j, slot)
                    store(j, slot)

                    @pl.when(j + 2 < n_chunks)
                    def _():
                        start(j + 2, slot)
                return carry

            lax.fori_loop(0, n_chunks // 2, pair, 0)

    return k(table, idx)


def _pt_body(f_ref, xp_ref, gg_ref, wq_ref, wkc_ref, wvc_ref,
             p1o_ref, p1c_ref, p1b_ref, p2_ref, p2b_ref, a1_ref, a1b_ref,
             a2_ref, a2b_ref, g_ref, b_ref, o_ref, *, bn, k, dim):
    f = f_ref[0]
    xp = xp_ref[0]
    gg = gg_ref[...]
    q3 = _dot(f, wq_ref[...])[:, None, :]
    kk = _dot(gg, wkc_ref[...]).reshape(bn, k, dim)
    v = _dot(gg, wvc_ref[...]).reshape(bn, k, dim)
    pe_own = _dot(xp, p1o_ref[...])[:, None, :]
    pe_nb = _dot(gg, p1c_ref[...]).reshape(bn, k, dim)
    peh = jnp.maximum(pe_own - pe_nb + p1b_ref[...][None], 0.0)
    pe = (_dot(peh.reshape(bn * k, dim), p2_ref[...])
          + p2b_ref[...]).reshape(bn, k, dim)
    s = q3 - kk + pe
    h = jnp.maximum(_dot(s.reshape(bn * k, dim), a1_ref[...])
                    + a1b_ref[...], 0.0)
    a = (_dot(h, a2_ref[...]) + a2b_ref[...]).reshape(bn, k, dim)
    mx = jnp.max(a, axis=1, keepdims=True)
    e = jnp.exp(a - mx)
    attn = e / jnp.sum(e, axis=1, keepdims=True)
    out = jnp.sum((v + pe) * attn, axis=1)
    o_ref[0] = _ln_in(out + f, g_ref[...], b_ref[...])


def _pt(feat, xp, gg, p, k):
    b_, n_, dim = feat.shape
    dg = gg.shape[1]
    bn = min(n_, 256)
    nb = n_ // bn
    body = functools.partial(_pt_body, bn=bn, k=k, dim=dim)
    row2 = lambda b, i: (b * nb + i, 0)
    wspec = pl.BlockSpec((dim, dim), lambda b, i: (0, 0))
    cspec = pl.BlockSpec((dg, dim), lambda b, i: (0, 0))
    bspec = pl.BlockSpec((1, dim), lambda b, i: (0, 0))
    z = jnp.zeros((dg, dim), _F32)
    wkc = z.at[16:16 + dim].set(p['wk'])
    wvc = z.at[16:16 + dim].set(p['wv'])
    p1c = z.at[:3].set(p['pe1_w'])
    p1o = jnp.pad(p['pe1_w'], ((0, 13), (0, 0)))
    return pl.pallas_call(
        body,
        grid=(b_, nb),
        in_specs=[
            pl.BlockSpec((1, bn, dim), lambda b, i: (b, i, 0)),
            pl.BlockSpec((1, bn, 16), lambda b, i: (b, i, 0)),
            pl.BlockSpec((bn * k, dg), row2),
            wspec, cspec, cspec,
            pl.BlockSpec((16, dim), lambda b, i: (0, 0)), cspec, bspec,
            wspec, bspec,
            wspec, bspec,
            wspec, bspec,
            bspec, bspec,
        ],
        out_specs=pl.BlockSpec((1, bn, dim), lambda b, i: (b, i, 0)),
        out_shape=jax.ShapeDtypeStruct((b_, n_, dim), _F32),
    )(feat, xp, gg, p['wq'], wkc, wvc,
      p1o, p1c, p['pe1_b'][None],
      p['pe2_w'], p['pe2_b'][None],
      p['am1_w'], p['am1_b'][None],
      p['am2_w'], p['am2_b'][None],
      p['ln_g'][None], p['ln_b'][None])


def _td_body(nx_ref, gg_ref, w1c_ref, w1x_ref, b1_ref,
             w2_ref, b2_ref, g_ref, b_ref, o_ref, *, bn, k, dout):
    nx = nx_ref[0]
    gg = gg_ref[...]
    hg = _dot(gg, w1c_ref[...]).reshape(bn, k, dout)
    ox = _dot(nx, w1x_ref[...])[:, None, :]
    h1 = jnp.maximum(hg - ox + b1_ref[...][None], 0.0)
    h2 = (_dot(h1.reshape(bn * k, dout), w2_ref[...])
          + b2_ref[...]).reshape(bn, k, dout)
    nf = jnp.max(h2, axis=1)
    o_ref[0] = _ln_in(nf, g_ref[...], b_ref[...])


def _td(nxp, gg, p, k, din):
    b_, np_, _ = nxp.shape
    dg = gg.shape[1]
    dout = p['w2'].shape[0]
    bn = min(np_, 256)
    nb = np_ // bn
    body = functools.partial(_td_body, bn=bn, k=k, dout=dout)
    row2 = lambda b, i: (b * nb + i, 0)
    w1x = jnp.pad(p['w1'][:3], ((0, 13), (0, 0)))
    w1c = jnp.zeros((dg, dout), _F32)
    w1c = w1c.at[:3].set(p['w1'][:3]).at[16:16 + din].set(p['w1'][3:])
    return pl.pallas_call(
        body,
        grid=(b_, nb),
        in_specs=[
            pl.BlockSpec((1, bn, 16), lambda b, i: (b, i, 0)),
            pl.BlockSpec((bn * k, dg), row2),
            pl.BlockSpec((dg, dout), lambda b, i: (0, 0)),
            pl.BlockSpec((16, dout), lambda b, i: (0, 0)),
            pl.BlockSpec((1, dout), lambda b, i: (0, 0)),
            pl.BlockSpec((dout, dout), lambda b, i: (0, 0)),
            pl.BlockSpec((1, dout), lambda b, i: (0, 0)),
            pl.BlockSpec((1, dout), lambda b, i: (0, 0)),
            pl.BlockSpec((1, dout), lambda b, i: (0, 0)),
        ],
        out_specs=pl.BlockSpec((1, bn, dout), lambda b, i: (b, i, 0)),
        out_shape=jax.ShapeDtypeStruct((b_, np_, dout), _F32),
    )(nxp, gg, w1c, w1x, p['b1'][None], p['w2'], p['b2'][None],
      p['ln_g'][None], p['ln_b'][None])


def _tu_body(hx_ref, lx_ref, fs_ref, fl_ref, w1s_ref, w1i_ref, b1_ref,
             w2_ref, b2_ref, g_ref, b_ref, o_ref):
    q = hx_ref[0]
    t = lx_ref[0]
    dd = _pair_dist(q, t)
    cio = lax.broadcasted_iota(_I32, dd.shape, 1)
    big = jnp.int32(2 ** 30)
    w = jnp.zeros(dd.shape, _F32)
    invsum = jnp.zeros((dd.shape[0], 1), _F32)
    for _ in range(3):
        mn = jnp.min(dd, axis=1, keepdims=True)
        cand = jnp.where(dd <= mn, cio, big)
        idx = jnp.min(cand, axis=1, keepdims=True)
        invj = 1.0 / (mn + 1e-8)
        sel = cio == idx
        w = w + jnp.where(sel, invj, 0.0)
        invsum = invsum + invj
        dd = jnp.where(sel, jnp.float32(jnp.inf), dd)
    w = w / invsum
    interp = _dot_hi(w, fl_ref[0])
    h = jnp.maximum(_dot(fs_ref[0], w1s_ref[...])
                    + _dot(interp, w1i_ref[...]) + b1_ref[...], 0.0)
    h2 = _dot(h, w2_ref[...]) + b2_ref[...]
    o_ref[0] = _ln_in(h2, g_ref[...], b_ref[...])


def _tu(hxp, lxp, fskip, flo, p):
    b_, nhi, _ = hxp.shape
    nlo = lxp.shape[1]
    dskip = fskip.shape[2]
    dlo = flo.shape[2]
    dout = p['w2'].shape[0]
    bn = min(nhi, 256)
    nb = nhi // bn
    return pl.pallas_call(
        _tu_body,
        grid=(b_, nb),
        in_specs=[
            pl.BlockSpec((1, bn, 16), lambda b, i: (b, i, 0)),
            pl.BlockSpec((1, nlo, 16), lambda b, i: (b, 0, 0)),
            pl.BlockSpec((1, bn, dskip), lambda b, i: (b, i, 0)),
            pl.BlockSpec((1, nlo, dlo), lambda b, i: (b, 0, 0)),
            pl.BlockSpec((dskip, dout), lambda b, i: (0, 0)),
            pl.BlockSpec((dlo, dout), lambda b, i: (0, 0)),
            pl.BlockSpec((1, dout), lambda b, i: (0, 0)),
            pl.BlockSpec((dout, dout), lambda b, i: (0, 0)),
            pl.BlockSpec((1, dout), lambda b, i: (0, 0)),
            pl.BlockSpec((1, dout), lambda b, i: (0, 0)),
            pl.BlockSpec((1, dout), lambda b, i: (0, 0)),
        ],
        out_specs=pl.BlockSpec((1, bn, dout), lambda b, i: (b, i, 0)),
        out_shape=jax.ShapeDtypeStruct((b_, nhi, dout), _F32),
    )(hxp, lxp, fskip, flo, p['w1'][:dskip], p['w1'][dskip:], p['b1'][None],
      p['w2'], p['b2'][None], p['ln_g'][None], p['ln_b'][None])


def _table(xp, feat):
    b_, n_, dim = feat.shape
    dg = ((16 + dim + 127) // 128) * 128
    t = jnp.concatenate(
        [xp, feat, jnp.zeros((b_, n_, dg - 16 - dim), _F32)], axis=-1)
    return t.reshape(b_ * n_, dg)


def kernel(x, params):
    p = params
    b_, n_, _ = x.shape
    xyz = x[..., :3]
    xp0 = jnp.pad(xyz, ((0, 0), (0, 0), (0, 13)))
    x8 = jnp.pad(x, ((0, 0), (0, 0), (0, 2)))
    stem_w = jnp.pad(p['stem_w'], ((0, 2), (0, 0)))
    f0 = _linear_ln(x8, stem_w, p['stem_b'][None],
                    p['stem_ln_g'][None], p['stem_ln_b'][None], True)

    idx0 = _knn(xp0, xp0, K_NN).reshape(-1)
    t0 = _table(xp0, f0)
    g0 = _gather(t0, idx0)
    f0p = _pt(f0, xp0, g0, p['pt0'], K_NN)

    fi1 = _fps(xyz, 1024)
    t0p = _table(xp0, f0p)
    xp1 = _gather(t0p, fi1)[:, :16].reshape(b_, 1024, 16)
    idxd1 = _knn(xp1, xp0, K_NN).reshape(-1)
    gd1 = _gather(t0p, idxd1)
    f1 = _td(xp1, gd1, p['td1'], K_NN, 64)

    idx1 = _knn(xp1, xp1, K_NN).reshape(-1)
    g1 = _gather(_table(xp1, f1), idx1)
    f1p = _pt(f1, xp1, g1, p['pt1'], K_NN)

    fi2 = _fps(xp1[..., :3], 256)
    t1p = _table(xp1, f1p)
    xp2 = _gather(t1p, fi2)[:, :16].reshape(b_, 256, 16)
    idxd2 = _knn(xp2, xp1, K_NN).reshape(-1)
    gd2 = _gather(t1p, idxd2)
    f2 = _td(xp2, gd2, p['td2'], K_NN, 128)

    idx2 = _knn(xp2, xp2, K_NN).reshape(-1)
    g2 = _gather(_table(xp2, f2), idx2)
    f2p = _pt(f2, xp2, g2, p['pt2'], K_NN)

    f1u = _tu(xp1, xp2, f1p, f2p, p['tu1'])
    g1u = _gather(_table(xp1, f1u), idx1)
    f1d = _pt(f1u, xp1, g1u, p['ptd1'], K_NN)

    f0u = _tu(xp0, xp1, f0p, f1d, p['tu2'])
    g0u = _gather(_table(xp0, f0u), idx0)
    f0d = _pt(f0u, xp0, g0u, p['ptd2'], K_NN)

    return _linear_ln(f0d, p['head_w'], p['head_b'][None],
                      p['head_ln_g'][None], p['head_ln_b'][None], False)


# speedup vs baseline: 11.5005x; 1.0135x over previous
import functools

import jax
import jax.numpy as jnp
from jax import lax
from jax.experimental import pallas as pl
from jax.experimental.pallas import tpu as pltpu
from jax.experimental.pallas import tpu_sc as plsc

B = 4
N0 = 4096
K_NN = 16

_F32 = jnp.float32
_I32 = jnp.int32


def _dot(a, b):
    return jnp.dot(a, b, preferred_element_type=_F32)


def _dot_nt(a, b):
    return lax.dot_general(a, b, (((1,), (1,)), ((), ())),
                           preferred_element_type=_F32)


def _dot_hi(a, b):
    return jnp.dot(a, b, preferred_element_type=_F32,
                   precision=lax.Precision.HIGHEST)


def _pair_dist(q, t):
    qt = _dot_nt(q, t)
    qsq = jnp.sum(q * q, axis=1, keepdims=True)
    tsq_row = lax.dot_general(jnp.ones((1, q.shape[1]), _F32), t * t,
                              (((1,), (1,)), ((), ())),
                              preferred_element_type=_F32,
                              precision=lax.Precision.HIGHEST)
    d2 = (qsq + tsq_row) - 2.0 * qt
    return jnp.sqrt(jnp.maximum(d2, 0.0))


def _ln_in(x, g, b, eps=1e-5):
    mu = jnp.mean(x, -1, keepdims=True)
    var = jnp.mean((x - mu) ** 2, -1, keepdims=True)
    return (x - mu) / jnp.sqrt(var + eps) * g + b


def _linear_ln_body(x_ref, w_ref, b_ref, g_ref, bb_ref, o_ref, *, relu_after):
    h = _dot(x_ref[0], w_ref[...]) + b_ref[...]
    h = _ln_in(h, g_ref[...], bb_ref[...])
    if relu_after:
        h = jnp.maximum(h, 0.0)
    o_ref[0] = h


def _linear_ln(x, w, bias, g, bb, relu_after):
    b_, n_, din = x.shape
    dout = w.shape[1]
    bl = min(n_, 2048)
    body = functools.partial(_linear_ln_body, relu_after=relu_after)
    return pl.pallas_call(
        body,
        grid=(b_, n_ // bl),
        in_specs=[
            pl.BlockSpec((1, bl, din), lambda b, i: (b, i, 0)),
            pl.BlockSpec((din, dout), lambda b, i: (0, 0)),
            pl.BlockSpec((1, dout), lambda b, i: (0, 0)),
            pl.BlockSpec((1, dout), lambda b, i: (0, 0)),
            pl.BlockSpec((1, dout), lambda b, i: (0, 0)),
        ],
        out_specs=pl.BlockSpec((1, bl, dout), lambda b, i: (b, i, 0)),
        out_shape=jax.ShapeDtypeStruct((b_, n_, dout), _F32),
    )(x, w, bias, g, bb)


def _knn_body(q_ref, t_ref, o_ref, *, k, nt):
    b = pl.program_id(0)
    q = q_ref[0]
    t = t_ref[0]
    dd = _pair_dist(q, t)
    half = nt // 2
    a = dd[:, :half]
    b2 = dd[:, half:]
    win = a <= b2
    m = jnp.where(win, a, b2)
    lo = jnp.where(win, b2, a)
    cio = lax.broadcasted_iota(_I32, (dd.shape[0], half), 1)
    colm = jnp.where(win, cio, cio + half)
    coll = jnp.where(win, cio + half, cio)
    big = jnp.int32(2 ** 30)
    inf = jnp.float32(jnp.inf)
    cols = []
    for _ in range(k):
        mn = jnp.min(m, axis=1, keepdims=True)
        cand = jnp.where(m <= mn, colm, big)
        c = jnp.min(cand, axis=1, keepdims=True)
        cols.append(c)
        sel = cand == c
        m = jnp.where(sel, lo, m)
        colm = jnp.where(sel, coll, colm)
        lo = jnp.where(sel, inf, lo)
    o_ref[0] = jnp.concatenate(cols, axis=1) + b * nt


def _knn(qxyz, txyz, k):
    b_, nq, _ = qxyz.shape
    nt = txyz.shape[1]
    bq = min(nq, 512)
    body = functools.partial(_knn_body, k=k, nt=nt)
    return pl.pallas_call(
        body,
        grid=(b_, nq // bq),
        in_specs=[
            pl.BlockSpec((1, bq, 16), lambda b, i: (b, i, 0)),
            pl.BlockSpec((1, nt, 16), lambda b, i: (b, 0, 0)),
        ],
        out_specs=pl.BlockSpec((1, bq, k), lambda b, i: (b, i, 0)),
        out_shape=jax.ShapeDtypeStruct((b_, nq, k), _I32),
    )(qxyz, txyz)


def _fps_body(x_ref, y_ref, z_ref, o_ref, *, npoint, n, r, nb):
    xs = x_ref[...]
    ys = y_ref[...]
    zs = z_ref[...]
    lin = (lax.broadcasted_iota(_I32, (nb, r, 128), 1) * 128
           + lax.broadcasted_iota(_I32, (nb, r, 128), 2))
    big = jnp.int32(2 ** 30)

    def red2(x, fn):
        return fn(fn(x, axis=2, keepdims=True), axis=1, keepdims=True)

    def body(t, carry):
        dist, far = carry
        for b in range(nb):
            o_ref[b, pl.ds(t, 1), :] = far[b] + b * n
        m1 = lin == far
        cx = red2(jnp.where(m1, xs, 0.0), jnp.sum)
        cy = red2(jnp.where(m1, ys, 0.0), jnp.sum)
        cz = red2(jnp.where(m1, zs, 0.0), jnp.sum)
        dx = xs - cx
        dy = ys - cy
        dz = zs - cz
        d2 = (dx * dx + dy * dy) + dz * dz
        dist = jnp.minimum(dist, d2)
        mx = red2(dist, jnp.max)
        far = red2(jnp.where(dist >= mx, lin, big), jnp.min)
        return dist, far

    dist0 = jnp.full((nb, r, 128), 1e10, _F32)
    far0 = jnp.zeros((nb, 1, 1), _I32)
    lax.fori_loop(0, npoint, body, (dist0, far0))


def _fps(xyz, npoint):
    b_, n_, _ = xyz.shape
    r = n_ // 128
    xs = xyz[..., 0].reshape(b_, r, 128)
    ys = xyz[..., 1].reshape(b_, r, 128)
    zs = xyz[..., 2].reshape(b_, r, 128)
    body = functools.partial(_fps_body, npoint=npoint, n=n_, r=r, nb=b_)
    out = pl.pallas_call(
        body,
        grid=(1,),
        in_specs=[pl.BlockSpec((b_, r, 128), lambda i: (0, 0, 0))] * 3,
        out_specs=pl.BlockSpec((b_, npoint, 1), lambda i: (0, 0, 0)),
        out_shape=jax.ShapeDtypeStruct((b_, npoint, 1), _I32),
    )(xs, ys, zs)
    return out.reshape(b_ * npoint)


def _gather(table, idx):
    m = idx.shape[0]
    d = table.shape[1]
    info = plsc.get_sparse_core_info()
    nw = info.num_cores * info.num_subcores
    rows_pw = m // nw
    chunk = min(128, rows_pw)
    n_chunks = rows_pw // chunk
    mesh = plsc.VectorSubcoreMesh(core_axis_name="c", subcore_axis_name="s")

    @functools.partial(
        pl.kernel, mesh=mesh,
        out_type=jax.ShapeDtypeStruct((m, d), _F32),
        scratch_types=[
            pltpu.VMEM((rows_pw,), _I32),
            pltpu.VMEM((2, chunk, d), _F32),
            pltpu.SemaphoreType.DMA,
            pltpu.SemaphoreType.DMA,
        ],
    )
    def k(table_hbm, idx_hbm, out_hbm, idx_v, rows_v, sem0, sem1):
        wid = lax.axis_index("s") * info.num_cores + lax.axis_index("c")
        base = wid * rows_pw
        pltpu.sync_copy(idx_hbm.at[pl.ds(base, rows_pw)], idx_v)
        sems = (sem0, sem1)

        def start(j, slot):
            pltpu.async_copy(
                table_hbm.at[idx_v.at[pl.ds(j * chunk, chunk)]],
                rows_v.at[slot], sems[slot])

        def wait(j, slot):
            pltpu.make_async_copy(
                table_hbm.at[idx_v.at[pl.ds(j * chunk, chunk)]],
                rows_v.at[slot], sems[slot]).wait()

        def store(j, slot):
            pltpu.sync_copy(rows_v.at[slot],
                            out_hbm.at[pl.ds(base + j * chunk, chunk)])

        if n_chunks == 1:
            start(0, 0)
            wait(0, 0)
            store(0, 0)
        else:
            start(0, 0)
            start(1, 1)

            def pair(g, carry):
                for slot in range(2):
                    j = g * 2 + slot
                    wait(j, slot)
                    store(j, slot)

                    @pl.when(j + 2 < n_chunks)
                    def _():
                        start(j + 2, slot)
                return carry

            lax.fori_loop(0, n_chunks // 2, pair, 0)

    return k(table, idx)


def _pt_body(f_ref, xp_ref, gg_ref, wq_ref, wkc_ref, wvc_ref,
             p1o_ref, p1c_ref, p1b_ref, p2_ref, p2b_ref, a1_ref, a1b_ref,
             a2_ref, a2b_ref, g_ref, b_ref, o_ref, *, bn, k, dim):
    f = f_ref[0]
    xp = xp_ref[0]
    gg = gg_ref[...]
    q3 = _dot(f, wq_ref[...])[:, None, :]
    kk = _dot(gg, wkc_ref[...]).reshape(bn, k, dim)
    v = _dot(gg, wvc_ref[...]).reshape(bn, k, dim)
    pe_own = _dot(xp, p1o_ref[...])[:, None, :]
    pe_nb = _dot(gg, p1c_ref[...]).reshape(bn, k, dim)
    peh = jnp.maximum(pe_own - pe_nb + p1b_ref[...][None], 0.0)
    pe = (_dot(peh.reshape(bn * k, dim), p2_ref[...])
          + p2b_ref[...]).reshape(bn, k, dim)
    s = q3 - kk + pe
    h = jnp.maximum(_dot(s.reshape(bn * k, dim), a1_ref[...])
                    + a1b_ref[...], 0.0)
    a = (_dot(h, a2_ref[...]) + a2b_ref[...]).reshape(bn, k, dim)
    mx = jnp.max(a, axis=1, keepdims=True)
    e = jnp.exp(a - mx)
    attn = e / jnp.sum(e, axis=1, keepdims=True)
    out = jnp.sum((v + pe) * attn, axis=1)
    o_ref[0] = _ln_in(out + f, g_ref[...], b_ref[...])


def _pt(feat, xp, gg, p, k):
    b_, n_, dim = feat.shape
    dg = gg.shape[1]
    bn = min(n_, 256)
    nb = n_ // bn
    body = functools.partial(_pt_body, bn=bn, k=k, dim=dim)
    row2 = lambda b, i: (b * nb + i, 0)
    wspec = pl.BlockSpec((dim, dim), lambda b, i: (0, 0))
    cspec = pl.BlockSpec((dg, dim), lambda b, i: (0, 0))
    bspec = pl.BlockSpec((1, dim), lambda b, i: (0, 0))
    z = jnp.zeros((dg, dim), _F32)
    wkc = z.at[16:16 + dim].set(p['wk'])
    wvc = z.at[16:16 + dim].set(p['wv'])
    p1c = z.at[:3].set(p['pe1_w'])
    p1o = jnp.pad(p['pe1_w'], ((0, 13), (0, 0)))
    return pl.pallas_call(
        body,
        grid=(b_, nb),
        in_specs=[
            pl.BlockSpec((1, bn, dim), lambda b, i: (b, i, 0)),
            pl.BlockSpec((1, bn, 16), lambda b, i: (b, i, 0)),
            pl.BlockSpec((bn * k, dg), row2),
            wspec, cspec, cspec,
            pl.BlockSpec((16, dim), lambda b, i: (0, 0)), cspec, bspec,
            wspec, bspec,
            wspec, bspec,
            wspec, bspec,
            bspec, bspec,
        ],
        out_specs=pl.BlockSpec((1, bn, dim), lambda b, i: (b, i, 0)),
        out_shape=jax.ShapeDtypeStruct((b_, n_, dim), _F32),
    )(feat, xp, gg, p['wq'], wkc, wvc,
      p1o, p1c, p['pe1_b'][None],
      p['pe2_w'], p['pe2_b'][None],
      p['am1_w'], p['am1_b'][None],
      p['am2_w'], p['am2_b'][None],
      p['ln_g'][None], p['ln_b'][None])


def _td_body(nx_ref, gg_ref, w1c_ref, w1x_ref, b1_ref,
             w2_ref, b2_ref, g_ref, b_ref, o_ref, *, bn, k, dout):
    nx = nx_ref[0]
    gg = gg_ref[...]
    hg = _dot(gg, w1c_ref[...]).reshape(bn, k, dout)
    ox = _dot(nx, w1x_ref[...])[:, None, :]
    h1 = jnp.maximum(hg - ox + b1_ref[...][None], 0.0)
    h2 = (_dot(h1.reshape(bn * k, dout), w2_ref[...])
          + b2_ref[...]).reshape(bn, k, dout)
    nf = jnp.max(h2, axis=1)
    o_ref[0] = _ln_in(nf, g_ref[...], b_ref[...])


def _td(nxp, gg, p, k, din):
    b_, np_, _ = nxp.shape
    dg = gg.shape[1]
    dout = p['w2'].shape[0]
    bn = min(np_, 256)
    nb = np_ // bn
    body = functools.partial(_td_body, bn=bn, k=k, dout=dout)
    row2 = lambda b, i: (b * nb + i, 0)
    w1x = jnp.pad(p['w1'][:3], ((0, 13), (0, 0)))
    w1c = jnp.zeros((dg, dout), _F32)
    w1c = w1c.at[:3].set(p['w1'][:3]).at[16:16 + din].set(p['w1'][3:])
    return pl.pallas_call(
        body,
        grid=(b_, nb),
        in_specs=[
            pl.BlockSpec((1, bn, 16), lambda b, i: (b, i, 0)),
            pl.BlockSpec((bn * k, dg), row2),
            pl.BlockSpec((dg, dout), lambda b, i: (0, 0)),
            pl.BlockSpec((16, dout), lambda b, i: (0, 0)),
            pl.BlockSpec((1, dout), lambda b, i: (0, 0)),
            pl.BlockSpec((dout, dout), lambda b, i: (0, 0)),
            pl.BlockSpec((1, dout), lambda b, i: (0, 0)),
            pl.BlockSpec((1, dout), lambda b, i: (0, 0)),
            pl.BlockSpec((1, dout), lambda b, i: (0, 0)),
        ],
        out_specs=pl.BlockSpec((1, bn, dout), lambda b, i: (b, i, 0)),
        out_shape=jax.ShapeDtypeStruct((b_, np_, dout), _F32),
    )(nxp, gg, w1c, w1x, p['b1'][None], p['w2'], p['b2'][None],
      p['ln_g'][None], p['ln_b'][None])


def _tu_body(hx_ref, lx_ref, fs_ref, fl_ref, w1s_ref, w1i_ref, b1_ref,
             w2_ref, b2_ref, g_ref, b_ref, o_ref):
    q = hx_ref[0]
    t = lx_ref[0]
    dd = _pair_dist(q, t)
    cio = lax.broadcasted_iota(_I32, dd.shape, 1)
    big = jnp.int32(2 ** 30)
    w = jnp.zeros(dd.shape, _F32)
    invsum = jnp.zeros((dd.shape[0], 1), _F32)
    for _ in range(3):
        mn = jnp.min(dd, axis=1, keepdims=True)
        cand = jnp.where(dd <= mn, cio, big)
        idx = jnp.min(cand, axis=1, keepdims=True)
        invj = 1.0 / (mn + 1e-8)
        sel = cio == idx
        w = w + jnp.where(sel, invj, 0.0)
        invsum = invsum + invj
        dd = jnp.where(sel, jnp.float32(jnp.inf), dd)
    w = w / invsum
    interp = _dot_hi(w, fl_ref[0])
    h = jnp.maximum(_dot(fs_ref[0], w1s_ref[...])
                    + _dot(interp, w1i_ref[...]) + b1_ref[...], 0.0)
    h2 = _dot(h, w2_ref[...]) + b2_ref[...]
    o_ref[0] = _ln_in(h2, g_ref[...], b_ref[...])


def _tu(hxp, lxp, fskip, flo, p):
    b_, nhi, _ = hxp.shape
    nlo = lxp.shape[1]
    dskip = fskip.shape[2]
    dlo = flo.shape[2]
    dout = p['w2'].shape[0]
    bn = min(nhi, 256)
    nb = nhi // bn
    return pl.pallas_call(
        _tu_body,
        grid=(b_, nb),
        in_specs=[
            pl.BlockSpec((1, bn, 16), lambda b, i: (b, i, 0)),
            pl.BlockSpec((1, nlo, 16), lambda b, i: (b, 0, 0)),
            pl.BlockSpec((1, bn, dskip), lambda b, i: (b, i, 0)),
            pl.BlockSpec((1, nlo, dlo), lambda b, i: (b, 0, 0)),
            pl.BlockSpec((dskip, dout), lambda b, i: (0, 0)),
            pl.BlockSpec((dlo, dout), lambda b, i: (0, 0)),
            pl.BlockSpec((1, dout), lambda b, i: (0, 0)),
            pl.BlockSpec((dout, dout), lambda b, i: (0, 0)),
            pl.BlockSpec((1, dout), lambda b, i: (0, 0)),
            pl.BlockSpec((1, dout), lambda b, i: (0, 0)),
            pl.BlockSpec((1, dout), lambda b, i: (0, 0)),
        ],
        out_specs=pl.BlockSpec((1, bn, dout), lambda b, i: (b, i, 0)),
        out_shape=jax.ShapeDtypeStruct((b_, nhi, dout), _F32),
    )(hxp, lxp, fskip, flo, p['w1'][:dskip], p['w1'][dskip:], p['b1'][None],
      p['w2'], p['b2'][None], p['ln_g'][None], p['ln_b'][None])


def _table(xp, feat):
    b_, n_, dim = feat.shape
    dg = ((16 + dim + 127) // 128) * 128
    t = jnp.concatenate(
        [xp, feat, jnp.zeros((b_, n_, dg - 16 - dim), _F32)], axis=-1)
    return t.reshape(b_ * n_, dg)


def kernel(x, params):
    p = params
    b_, n_, _ = x.shape
    xyz = x[..., :3]
    xp0 = jnp.pad(xyz, ((0, 0), (0, 0), (0, 13)))
    x8 = jnp.pad(x, ((0, 0), (0, 0), (0, 2)))
    stem_w = jnp.pad(p['stem_w'], ((0, 2), (0, 0)))
    f0 = _linear_ln(x8, stem_w, p['stem_b'][None],
                    p['stem_ln_g'][None], p['stem_ln_b'][None], True)

    idx0 = _knn(xp0, xp0, K_NN).reshape(-1)
    t0 = _table(xp0, f0)
    g0 = _gather(t0, idx0)
    f0p = _pt(f0, xp0, g0, p['pt0'], K_NN)

    fi1 = _fps(xyz, 1024)
    t0p = _table(xp0, f0p)
    xp1 = _gather(t0p, fi1)[:, :16].reshape(b_, 1024, 16)
    idxd1 = _knn(xp1, xp0, K_NN).reshape(-1)
    gd1 = _gather(t0p, idxd1)
    f1 = _td(xp1, gd1, p['td1'], K_NN, 64)

    idx1 = _knn(xp1, xp1, K_NN).reshape(-1)
    g1 = _gather(_table(xp1, f1), idx1)
    f1p = _pt(f1, xp1, g1, p['pt1'], K_NN)

    fi2 = _fps(xp1[..., :3], 256)
    t1p = _table(xp1, f1p)
    xp2 = _gather(t1p, fi2)[:, :16].reshape(b_, 256, 16)
    idxd2 = _knn(xp2, xp1, K_NN).reshape(-1)
    gd2 = _gather(t1p, idxd2)
    f2 = _td(xp2, gd2, p['td2'], K_NN, 128)

    idx2 = _knn(xp2, xp2, K_NN).reshape(-1)
    g2 = _gather(_table(xp2, f2), idx2)
    f2p = _pt(f2, xp2, g2, p['pt2'], K_NN)

    f1u = _tu(xp1, xp2, f1p, f2p, p['tu1'])
    g1u = _gather(_table(xp1, f1u), idx1)
    f1d = _pt(f1u, xp1, g1u, p['ptd1'], K_NN)

    f0u = _tu(xp0, xp1, f0p, f1d, p['tu2'])
    g0u = _gather(_table(xp0, f0u), idx0)
    f0d = _pt(f0u, xp0, g0u, p['ptd2'], K_NN)

    return _linear_ln(f0d, p['head_w'], p['head_b'][None],
                      p['head_ln_g'][None], p['head_ln_b'][None], False)


# speedup vs baseline: 13.1704x; 1.1452x over previous
import functools

import jax
import jax.numpy as jnp
from jax import lax
from jax.experimental import pallas as pl
from jax.experimental.pallas import tpu as pltpu
from jax.experimental.pallas import tpu_sc as plsc

B = 4
N0 = 4096
K_NN = 16

_F32 = jnp.float32
_I32 = jnp.int32


def _dot(a, b):
    return jnp.dot(a, b, preferred_element_type=_F32)


def _dot_nt(a, b):
    return lax.dot_general(a, b, (((1,), (1,)), ((), ())),
                           preferred_element_type=_F32)


def _dot_hi(a, b):
    return jnp.dot(a, b, preferred_element_type=_F32,
                   precision=lax.Precision.HIGHEST)


def _pair_dist(q, t, tsq_row):
    qt = _dot_nt(q, t)
    qsq = jnp.sum(q * q, axis=1, keepdims=True)
    d2 = (qsq + tsq_row) - 2.0 * qt
    return jnp.sqrt(jnp.maximum(d2, 0.0))


def _tsq_body(t_ref, o_ref):
    t = t_ref[0]
    o_ref[0] = lax.dot_general(jnp.ones((1, t.shape[1]), _F32), t * t,
                               (((1,), (1,)), ((), ())),
                               preferred_element_type=_F32,
                               precision=lax.Precision.HIGHEST)


def _tsq(txyz):
    b_, nt, _ = txyz.shape
    return pl.pallas_call(
        _tsq_body,
        grid=(b_,),
        in_specs=[pl.BlockSpec((1, nt, 16), lambda b: (b, 0, 0))],
        out_specs=pl.BlockSpec((1, 1, nt), lambda b: (b, 0, 0)),
        out_shape=jax.ShapeDtypeStruct((b_, 1, nt), _F32),
    )(txyz)


def _ln_in(x, g, b, eps=1e-5):
    mu = jnp.mean(x, -1, keepdims=True)
    var = jnp.mean((x - mu) ** 2, -1, keepdims=True)
    return (x - mu) / jnp.sqrt(var + eps) * g + b


def _linear_ln_body(x_ref, w_ref, b_ref, g_ref, bb_ref, o_ref, *, relu_after):
    h = _dot(x_ref[0], w_ref[...]) + b_ref[...]
    h = _ln_in(h, g_ref[...], bb_ref[...])
    if relu_after:
        h = jnp.maximum(h, 0.0)
    o_ref[0] = h


def _linear_ln(x, w, bias, g, bb, relu_after):
    b_, n_, din = x.shape
    dout = w.shape[1]
    bl = min(n_, 2048)
    body = functools.partial(_linear_ln_body, relu_after=relu_after)
    return pl.pallas_call(
        body,
        grid=(b_, n_ // bl),
        in_specs=[
            pl.BlockSpec((1, bl, din), lambda b, i: (b, i, 0)),
            pl.BlockSpec((din, dout), lambda b, i: (0, 0)),
            pl.BlockSpec((1, dout), lambda b, i: (0, 0)),
            pl.BlockSpec((1, dout), lambda b, i: (0, 0)),
            pl.BlockSpec((1, dout), lambda b, i: (0, 0)),
        ],
        out_specs=pl.BlockSpec((1, bl, dout), lambda b, i: (b, i, 0)),
        out_shape=jax.ShapeDtypeStruct((b_, n_, dout), _F32),
    )(x, w, bias, g, bb)


def _knn_body(q_ref, t_ref, ts_ref, o_ref, *, k, nt):
    b = pl.program_id(0)
    q = q_ref[0]
    t = t_ref[0]
    dd = _pair_dist(q, t, ts_ref[0])
    half = nt // 2
    a = dd[:, :half]
    b2 = dd[:, half:]
    win = a <= b2
    m = jnp.where(win, a, b2)
    lo = jnp.where(win, b2, a)
    cio = lax.broadcasted_iota(_I32, (dd.shape[0], half), 1)
    colm = jnp.where(win, cio, cio + half)
    coll = jnp.where(win, cio + half, cio)
    big = jnp.int32(2 ** 30)
    inf = jnp.float32(jnp.inf)
    cols = []
    for _ in range(k):
        mn = jnp.min(m, axis=1, keepdims=True)
        cand = jnp.where(m <= mn, colm, big)
        c = jnp.min(cand, axis=1, keepdims=True)
        cols.append(c)
        sel = cand == c
        m = jnp.where(sel, lo, m)
        colm = jnp.where(sel, coll, colm)
        lo = jnp.where(sel, inf, lo)
    o_ref[0] = jnp.concatenate(cols, axis=1) + b * nt


def _knn(qxyz, txyz, tsq, k):
    b_, nq, _ = qxyz.shape
    nt = txyz.shape[1]
    bq = min(nq, 512)
    body = functools.partial(_knn_body, k=k, nt=nt)
    return pl.pallas_call(
        body,
        grid=(b_, nq // bq),
        in_specs=[
            pl.BlockSpec((1, bq, 16), lambda b, i: (b, i, 0)),
            pl.BlockSpec((1, nt, 16), lambda b, i: (b, 0, 0)),
            pl.BlockSpec((1, 1, nt), lambda b, i: (b, 0, 0)),
        ],
        out_specs=pl.BlockSpec((1, bq, k), lambda b, i: (b, i, 0)),
        out_shape=jax.ShapeDtypeStruct((b_, nq, k), _I32),
    )(qxyz, txyz, tsq)


def _fps_body(x_ref, y_ref, z_ref, o_ref, *, npoint, n, r, nb):
    xs = x_ref[...]
    ys = y_ref[...]
    zs = z_ref[...]
    lin = (lax.broadcasted_iota(_I32, (nb, r, 128), 1) * 128
           + lax.broadcasted_iota(_I32, (nb, r, 128), 2))
    big = jnp.int32(2 ** 30)

    def red2(x, fn):
        return fn(fn(x, axis=2, keepdims=True), axis=1, keepdims=True)

    def body(t, carry):
        dist, far = carry
        for b in range(nb):
            o_ref[b, pl.ds(t, 1), :] = far[b] + b * n
        m1 = lin == far
        cx = red2(jnp.where(m1, xs, 0.0), jnp.sum)
        cy = red2(jnp.where(m1, ys, 0.0), jnp.sum)
        cz = red2(jnp.where(m1, zs, 0.0), jnp.sum)
        dx = xs - cx
        dy = ys - cy
        dz = zs - cz
        d2 = (dx * dx + dy * dy) + dz * dz
        dist = jnp.minimum(dist, d2)
        mx = red2(dist, jnp.max)
        far = red2(jnp.where(dist >= mx, lin, big), jnp.min)
        return dist, far

    dist0 = jnp.full((nb, r, 128), 1e10, _F32)
    far0 = jnp.zeros((nb, 1, 1), _I32)
    lax.fori_loop(0, npoint, body, (dist0, far0))


def _fps(xyz, npoint):
    b_, n_, _ = xyz.shape
    r = n_ // 128
    xs = xyz[..., 0].reshape(b_, r, 128)
    ys = xyz[..., 1].reshape(b_, r, 128)
    zs = xyz[..., 2].reshape(b_, r, 128)
    body = functools.partial(_fps_body, npoint=npoint, n=n_, r=r, nb=b_)
    out = pl.pallas_call(
        body,
        grid=(1,),
        in_specs=[pl.BlockSpec((b_, r, 128), lambda i: (0, 0, 0))] * 3,
        out_specs=pl.BlockSpec((b_, npoint, 1), lambda i: (0, 0, 0)),
        out_shape=jax.ShapeDtypeStruct((b_, npoint, 1), _I32),
    )(xs, ys, zs)
    return out.reshape(b_ * npoint)


def _gather(table, idx):
    m = idx.shape[0]
    d = table.shape[1]
    info = plsc.get_sparse_core_info()
    nw = info.num_cores * info.num_subcores
    rows_pw = m // nw
    chunk = min(128, rows_pw)
    n_chunks = rows_pw // chunk
    mesh = plsc.VectorSubcoreMesh(core_axis_name="c", subcore_axis_name="s")

    @functools.partial(
        pl.kernel, mesh=mesh,
        out_type=jax.ShapeDtypeStruct((m, d), _F32),
        scratch_types=[
            pltpu.VMEM((rows_pw,), _I32),
            pltpu.VMEM((2, chunk, d), _F32),
            pltpu.SemaphoreType.DMA,
            pltpu.SemaphoreType.DMA,
        ],
    )
    def k(table_hbm, idx_hbm, out_hbm, idx_v, rows_v, sem0, sem1):
        wid = lax.axis_index("s") * info.num_cores + lax.axis_index("c")
        base = wid * rows_pw
        pltpu.sync_copy(idx_hbm.at[pl.ds(base, rows_pw)], idx_v)
        sems = (sem0, sem1)

        def start(j, slot):
            pltpu.async_copy(
                table_hbm.at[idx_v.at[pl.ds(j * chunk, chunk)]],
                rows_v.at[slot], sems[slot])

        def wait(j, slot):
            pltpu.make_async_copy(
                table_hbm.at[idx_v.at[pl.ds(j * chunk, chunk)]],
                rows_v.at[slot], sems[slot]).wait()

        def store(j, slot):
            pltpu.sync_copy(rows_v.at[slot],
                            out_hbm.at[pl.ds(base + j * chunk, chunk)])

        if n_chunks == 1:
            start(0, 0)
            wait(0, 0)
            store(0, 0)
        else:
            start(0, 0)
            start(1, 1)

            def pair(g, carry):
                for slot in range(2):
                    j = g * 2 + slot
                    wait(j, slot)
                    store(j, slot)

                    @pl.when(j + 2 < n_chunks)
                    def _():
                        start(j + 2, slot)
                return carry

            lax.fori_loop(0, n_chunks // 2, pair, 0)

    return k(table, idx)


def _pt_body(f_ref, xp_ref, gg_ref, wq_ref, wkc_ref, wvc_ref,
             p1o_ref, p1c_ref, p1b_ref, p2_ref, p2b_ref, a1_ref, a1b_ref,
             a2_ref, a2b_ref, g_ref, b_ref, o_ref, *, bn, k, dim):
    f = f_ref[0]
    xp = xp_ref[0]
    gg = gg_ref[...]
    q3 = _dot(f, wq_ref[...])[:, None, :]
    kk = _dot(gg, wkc_ref[...]).reshape(bn, k, dim)
    v = _dot(gg, wvc_ref[...]).reshape(bn, k, dim)
    pe_own = _dot(xp, p1o_ref[...])[:, None, :]
    pe_nb = _dot(gg, p1c_ref[...]).reshape(bn, k, dim)
    peh = jnp.maximum(pe_own - pe_nb + p1b_ref[...][None], 0.0)
    pe = (_dot(peh.reshape(bn * k, dim), p2_ref[...])
          + p2b_ref[...]).reshape(bn, k, dim)
    s = q3 - kk + pe
    h = jnp.maximum(_dot(s.reshape(bn * k, dim), a1_ref[...])
                    + a1b_ref[...], 0.0)
    a = (_dot(h, a2_ref[...]) + a2b_ref[...]).reshape(bn, k, dim)
    mx = jnp.max(a, axis=1, keepdims=True)
    e = jnp.exp(a - mx)
    attn = e / jnp.sum(e, axis=1, keepdims=True)
    out = jnp.sum((v + pe) * attn, axis=1)
    o_ref[0] = _ln_in(out + f, g_ref[...], b_ref[...])


def _pt(feat, xp, gg, p, k):
    b_, n_, dim = feat.shape
    dg = gg.shape[1]
    bn = min(n_, 256)
    nb = n_ // bn
    body = functools.partial(_pt_body, bn=bn, k=k, dim=dim)
    row2 = lambda b, i: (b * nb + i, 0)
    wspec = pl.BlockSpec((dim, dim), lambda b, i: (0, 0))
    cspec = pl.BlockSpec((dg, dim), lambda b, i: (0, 0))
    bspec = pl.BlockSpec((1, dim), lambda b, i: (0, 0))
    z = jnp.zeros((dg, dim), _F32)
    wkc = z.at[16:16 + dim].set(p['wk'])
    wvc = z.at[16:16 + dim].set(p['wv'])
    p1c = z.at[:3].set(p['pe1_w'])
    p1o = jnp.pad(p['pe1_w'], ((0, 13), (0, 0)))
    return pl.pallas_call(
        body,
        grid=(b_, nb),
        in_specs=[
            pl.BlockSpec((1, bn, dim), lambda b, i: (b, i, 0)),
            pl.BlockSpec((1, bn, 16), lambda b, i: (b, i, 0)),
            pl.BlockSpec((bn * k, dg), row2),
            wspec, cspec, cspec,
            pl.BlockSpec((16, dim), lambda b, i: (0, 0)), cspec, bspec,
            wspec, bspec,
            wspec, bspec,
            wspec, bspec,
            bspec, bspec,
        ],
        out_specs=pl.BlockSpec((1, bn, dim), lambda b, i: (b, i, 0)),
        out_shape=jax.ShapeDtypeStruct((b_, n_, dim), _F32),
    )(feat, xp, gg, p['wq'], wkc, wvc,
      p1o, p1c, p['pe1_b'][None],
      p['pe2_w'], p['pe2_b'][None],
      p['am1_w'], p['am1_b'][None],
      p['am2_w'], p['am2_b'][None],
      p['ln_g'][None], p['ln_b'][None])


def _td_body(nx_ref, gg_ref, w1c_ref, w1x_ref, b1_ref,
             w2_ref, b2_ref, g_ref, b_ref, o_ref, *, bn, k, dout):
    nx = nx_ref[0]
    gg = gg_ref[...]
    hg = _dot(gg, w1c_ref[...]).reshape(bn, k, dout)
    ox = _dot(nx, w1x_ref[...])[:, None, :]
    h1 = jnp.maximum(hg - ox + b1_ref[...][None], 0.0)
    h2 = (_dot(h1.reshape(bn * k, dout), w2_ref[...])
          + b2_ref[...]).reshape(bn, k, dout)
    nf = jnp.max(h2, axis=1)
    o_ref[0] = _ln_in(nf, g_ref[...], b_ref[...])


def _td(nxp, gg, p, k, din):
    b_, np_, _ = nxp.shape
    dg = gg.shape[1]
    dout = p['w2'].shape[0]
    bn = min(np_, 256)
    nb = np_ // bn
    body = functools.partial(_td_body, bn=bn, k=k, dout=dout)
    row2 = lambda b, i: (b * nb + i, 0)
    w1x = jnp.pad(p['w1'][:3], ((0, 13), (0, 0)))
    w1c = jnp.zeros((dg, dout), _F32)
    w1c = w1c.at[:3].set(p['w1'][:3]).at[16:16 + din].set(p['w1'][3:])
    return pl.pallas_call(
        body,
        grid=(b_, nb),
        in_specs=[
            pl.BlockSpec((1, bn, 16), lambda b, i: (b, i, 0)),
            pl.BlockSpec((bn * k, dg), row2),
            pl.BlockSpec((dg, dout), lambda b, i: (0, 0)),
            pl.BlockSpec((16, dout), lambda b, i: (0, 0)),
            pl.BlockSpec((1, dout), lambda b, i: (0, 0)),
            pl.BlockSpec((dout, dout), lambda b, i: (0, 0)),
            pl.BlockSpec((1, dout), lambda b, i: (0, 0)),
            pl.BlockSpec((1, dout), lambda b, i: (0, 0)),
            pl.BlockSpec((1, dout), lambda b, i: (0, 0)),
        ],
        out_specs=pl.BlockSpec((1, bn, dout), lambda b, i: (b, i, 0)),
        out_shape=jax.ShapeDtypeStruct((b_, np_, dout), _F32),
    )(nxp, gg, w1c, w1x, p['b1'][None], p['w2'], p['b2'][None],
      p['ln_g'][None], p['ln_b'][None])


def _tu_body(hx_ref, lx_ref, ts_ref, fs_ref, fl_ref, w1s_ref, w1i_ref, b1_ref,
             w2_ref, b2_ref, g_ref, b_ref, o_ref):
    q = hx_ref[0]
    t = lx_ref[0]
    dd = _pair_dist(q, t, ts_ref[0])
    cio = lax.broadcasted_iota(_I32, dd.shape, 1)
    big = jnp.int32(2 ** 30)
    w = jnp.zeros(dd.shape, _F32)
    invsum = jnp.zeros((dd.shape[0], 1), _F32)
    for _ in range(3):
        mn = jnp.min(dd, axis=1, keepdims=True)
        cand = jnp.where(dd <= mn, cio, big)
        idx = jnp.min(cand, axis=1, keepdims=True)
        invj = 1.0 / (mn + 1e-8)
        sel = cio == idx
        w = w + jnp.where(sel, invj, 0.0)
        invsum = invsum + invj
        dd = jnp.where(sel, jnp.float32(jnp.inf), dd)
    w = w / invsum
    interp = _dot_hi(w, fl_ref[0])
    h = jnp.maximum(_dot(fs_ref[0], w1s_ref[...])
                    + _dot(interp, w1i_ref[...]) + b1_ref[...], 0.0)
    h2 = _dot(h, w2_ref[...]) + b2_ref[...]
    o_ref[0] = _ln_in(h2, g_ref[...], b_ref[...])


def _tu(hxp, lxp, tsq, fskip, flo, p):
    b_, nhi, _ = hxp.shape
    nlo = lxp.shape[1]
    dskip = fskip.shape[2]
    dlo = flo.shape[2]
    dout = p['w2'].shape[0]
    bn = min(nhi, 256)
    nb = nhi // bn
    return pl.pallas_call(
        _tu_body,
        grid=(b_, nb),
        in_specs=[
            pl.BlockSpec((1, bn, 16), lambda b, i: (b, i, 0)),
            pl.BlockSpec((1, nlo, 16), lambda b, i: (b, 0, 0)),
            pl.BlockSpec((1, 1, nlo), lambda b, i: (b, 0, 0)),
            pl.BlockSpec((1, bn, dskip), lambda b, i: (b, i, 0)),
            pl.BlockSpec((1, nlo, dlo), lambda b, i: (b, 0, 0)),
            pl.BlockSpec((dskip, dout), lambda b, i: (0, 0)),
            pl.BlockSpec((dlo, dout), lambda b, i: (0, 0)),
            pl.BlockSpec((1, dout), lambda b, i: (0, 0)),
            pl.BlockSpec((dout, dout), lambda b, i: (0, 0)),
            pl.BlockSpec((1, dout), lambda b, i: (0, 0)),
            pl.BlockSpec((1, dout), lambda b, i: (0, 0)),
            pl.BlockSpec((1, dout), lambda b, i: (0, 0)),
        ],
        out_specs=pl.BlockSpec((1, bn, dout), lambda b, i: (b, i, 0)),
        out_shape=jax.ShapeDtypeStruct((b_, nhi, dout), _F32),
    )(hxp, lxp, tsq, fskip, flo, p['w1'][:dskip], p['w1'][dskip:],
      p['b1'][None], p['w2'], p['b2'][None],
      p['ln_g'][None], p['ln_b'][None])


def _table(xp, feat):
    b_, n_, dim = feat.shape
    dg = ((16 + dim + 127) // 128) * 128
    t = jnp.concatenate(
        [xp, feat, jnp.zeros((b_, n_, dg - 16 - dim), _F32)], axis=-1)
    return t.reshape(b_ * n_, dg)


def kernel(x, params):
    p = params
    b_, n_, _ = x.shape
    xyz = x[..., :3]
    xp0 = jnp.pad(xyz, ((0, 0), (0, 0), (0, 13)))
    x8 = jnp.pad(x, ((0, 0), (0, 0), (0, 2)))
    stem_w = jnp.pad(p['stem_w'], ((0, 2), (0, 0)))
    f0 = _linear_ln(x8, stem_w, p['stem_b'][None],
                    p['stem_ln_g'][None], p['stem_ln_b'][None], True)

    tsq0 = _tsq(xp0)
    idx0 = _knn(xp0, xp0, tsq0, K_NN).reshape(-1)
    t0 = _table(xp0, f0)
    g0 = _gather(t0, idx0)
    f0p = _pt(f0, xp0, g0, p['pt0'], K_NN)

    fi1 = _fps(xyz, 1024)
    t0p = _table(xp0, f0p)
    xp1 = _gather(t0p, fi1)[:, :16].reshape(b_, 1024, 16)
    idxd1 = _knn(xp1, xp0, tsq0, K_NN).reshape(-1)
    gd1 = _gather(t0p, idxd1)
    f1 = _td(xp1, gd1, p['td1'], K_NN, 64)

    tsq1 = _tsq(xp1)
    idx1 = _knn(xp1, xp1, tsq1, K_NN).reshape(-1)
    g1 = _gather(_table(xp1, f1), idx1)
    f1p = _pt(f1, xp1, g1, p['pt1'], K_NN)

    fi2 = _fps(xp1[..., :3], 256)
    t1p = _table(xp1, f1p)
    xp2 = _gather(t1p, fi2)[:, :16].reshape(b_, 256, 16)
    idxd2 = _knn(xp2, xp1, tsq1, K_NN).reshape(-1)
    gd2 = _gather(t1p, idxd2)
    f2 = _td(xp2, gd2, p['td2'], K_NN, 128)

    tsq2 = _tsq(xp2)
    idx2 = _knn(xp2, xp2, tsq2, K_NN).reshape(-1)
    g2 = _gather(_table(xp2, f2), idx2)
    f2p = _pt(f2, xp2, g2, p['pt2'], K_NN)

    f1u = _tu(xp1, xp2, tsq2, f1p, f2p, p['tu1'])
    g1u = _gather(_table(xp1, f1u), idx1)
    f1d = _pt(f1u, xp1, g1u, p['ptd1'], K_NN)

    f0u = _tu(xp0, xp1, tsq1, f0p, f1d, p['tu2'])
    g0u = _gather(_table(xp0, f0u), idx0)
    f0d = _pt(f0u, xp0, g0u, p['ptd2'], K_NN)

    return _linear_ln(f0d, p['head_w'], p['head_b'][None],
                      p['head_ln_g'][None], p['head_ln_b'][None], False)
